# baseline (speedup 1.0000x reference)
import functools

import jax
import jax.numpy as jnp
import numpy as np
from jax import lax
from jax.experimental import pallas as pl
from jax.experimental.pallas import tpu as pltpu
from jax.experimental.pallas import tpu_sc as plsc

N = 10000
E = 320000
D = 128
HID = 128
OUT_DIM = 64
N_LAYERS = 3
NUM_SUB = 1000
NUM_GRAPHS = 16

NPAD = 10112
NBLK = NPAD // 128
NW = 32
EPW = 10240
EPAD = NW * EPW
CH = EPW // 128
NBUF = 2
ROWS_PER_TILE = NPAD // 16
LOGIT_MIN_SCORE = float(np.log(0.3 / 0.7))
SUBPAD = 1024
FEAT = 512

_mesh = plsc.VectorSubcoreMesh(core_axis_name="c", subcore_axis_name="s")


def _k1_score(x3_ref, w_ref, xw_ref, t_ref):
    x3 = x3_ref[...]
    w = w_ref[...]
    xw = jnp.sum(x3 * w[None, :, :], axis=2)
    flat = (lax.broadcasted_iota(jnp.int32, (NBLK, 128), 0) * 128
            + lax.broadcasted_iota(jnp.int32, (NBLK, 128), 1))
    xw_ref[...] = jnp.where(flat < N, xw, -1e30)
    nrm = jnp.sqrt(jnp.sum(w * w)) + 1e-12
    t_ref[...] = jnp.full((1, 128), LOGIT_MIN_SCORE, jnp.float32) * nrm


def _k2_mask(xw_hbm, t_hbm, src_hbm, dst_hbm, dstm_out, touch_out,
             xw_v, t_v, src_v, dst_v, dstm_v, touch_v):
    c = lax.axis_index("c")
    s = lax.axis_index("s")
    w = c * 16 + s
    base = w * EPW
    pltpu.sync_copy(xw_hbm, xw_v)
    pltpu.sync_copy(t_hbm, t_v)
    pltpu.sync_copy(src_hbm.at[pl.ds(base, EPW)], src_v)
    pltpu.sync_copy(dst_hbm.at[pl.ds(base, EPW)], dst_v)
    t16 = t_v[...]
    ones = jnp.full((16,), 1.0, jnp.float32)

    def zbody(i, carry):
        touch_v[pl.ds(i * 16, 16)] = jnp.zeros((16,), jnp.float32)
        return carry

    lax.fori_loop(0, NPAD // 16, zbody, 0)

    def body(i, carry):
        sl = pl.ds(i * 16, 16)
        s16 = src_v[sl]
        d16 = dst_v[sl]
        z = plsc.load_gather(xw_v, [s16]) + plsc.load_gather(xw_v, [d16])
        keep = z > t16
        dstm_v[sl] = jnp.where(keep, d16, N + (d16 & 63))
        plsc.store_scatter(touch_v, [s16], ones, mask=keep)
        plsc.store_scatter(touch_v, [d16], ones, mask=keep)
        return carry

    lax.fori_loop(0, EPW // 16, body, 0)
    pltpu.sync_copy(dstm_v, dstm_out.at[w])
    pltpu.sync_copy(touch_v, touch_out.at[w])


def _k3_msg(h_hbm, src3_hbm, dst3_hbm, zeros_hbm, out_hbm,
            sidx_v, didx_v, stage_v, acc_sh, gsem, ssem, isem):
    c = lax.axis_index("c")
    s = lax.axis_index("s")
    w = c * 16 + s
    rows = pl.ds(s * ROWS_PER_TILE, ROWS_PER_TILE)
    pltpu.sync_copy(zeros_hbm, acc_sh.at[rows])
    pltpu.sync_copy(src3_hbm.at[w], sidx_v)
    plsc.subcore_barrier()

    for b in range(NBUF):
        pltpu.async_copy(h_hbm.at[sidx_v.at[b]], stage_v.at[b], gsem.at[b])
        pltpu.async_copy(dst3_hbm.at[w].at[b], didx_v.at[b], isem.at[b])

    def body(t, carry):
        for b in range(NBUF):
            cc = t * NBUF + b
            pltpu.make_async_copy(
                h_hbm.at[sidx_v.at[cc]], stage_v.at[b], gsem.at[b]).wait()
            pltpu.make_async_copy(
                dst3_hbm.at[w].at[cc], didx_v.at[b], isem.at[b]).wait()
            pltpu.async_copy(stage_v.at[b], acc_sh.at[didx_v.at[b]],
                             ssem.at[b], add=True)
        for b in range(NBUF):
            cn = t * NBUF + b + NBUF
            pltpu.make_async_copy(
                stage_v.at[b], acc_sh.at[didx_v.at[b]], ssem.at[b]).wait()

            @pl.when(cn < CH)
            def _():
                pltpu.async_copy(h_hbm.at[sidx_v.at[cn]], stage_v.at[b],
                                 gsem.at[b])
                pltpu.async_copy(dst3_hbm.at[w].at[cn], didx_v.at[b],
                                 isem.at[b])
        return carry

    lax.fori_loop(0, CH // NBUF, body, 0)
    plsc.subcore_barrier()
    pltpu.sync_copy(acc_sh.at[rows], out_hbm.at[c].at[rows])


def _k4_mlp(h_ref, m0_ref, m1_ref, eps_ref, w1_ref, b1_ref, w2_ref, b2_ref,
            o_ref):
    agg = h_ref[...] * eps_ref[...] + m0_ref[...] + m1_ref[...]
    hh = jnp.maximum(
        jnp.dot(agg, w1_ref[...], preferred_element_type=jnp.float32)
        + b1_ref[...], 0.0)
    o_ref[...] = (jnp.dot(hh, w2_ref[...], preferred_element_type=jnp.float32)
                  + b2_ref[...])


def _k5_pool(b3_ref, t_ref, h1_ref, h2_ref, h3_ref, o_ref, x_scr):
    i = pl.program_id(0)
    batch = b3_ref[...].reshape(1, 128)
    m = (jnp.sum(t_ref[...], axis=0, keepdims=True) > 0.0).astype(
        jnp.bfloat16)
    seg = lax.broadcasted_iota(jnp.int32, (SUBPAD, 128), 0)
    sm = ((seg == batch).astype(jnp.float32).astype(jnp.bfloat16)
          * m)
    x_scr[:, 0:128] = h1_ref[...].astype(jnp.bfloat16)
    x_scr[:, 128:256] = h2_ref[...].astype(jnp.bfloat16)
    x_scr[:, 256:384] = h3_ref[...].astype(jnp.bfloat16)
    lane = lax.broadcasted_iota(jnp.int32, (128, 128), 1)
    x_scr[:, 384:512] = (lane == 0).astype(jnp.float32).astype(jnp.bfloat16)

    @pl.when(i == 0)
    def _():
        o_ref[...] = jnp.zeros((SUBPAD, FEAT), jnp.float32)

    o_ref[...] += jnp.dot(sm, x_scr[...],
                          preferred_element_type=jnp.float32)


def _k6_final(sums_ref, s2g_ref, w1_ref, b1_ref, w2_ref, b2_ref, o_ref):
    sums = sums_ref[...]
    s2g = s2g_ref[...].reshape(1, SUBPAD)
    cnt0 = sums[:, 384:512]
    ones_j = jnp.ones((128, 128), jnp.float32)
    den = jnp.maximum(
        jnp.dot(cnt0, ones_j, preferred_element_type=jnp.float32), 1.0)
    gmat = (lax.broadcasted_iota(jnp.int32, (NUM_GRAPHS, SUBPAD), 0)
            == s2g).astype(jnp.float32)
    g1 = jnp.dot(gmat, sums[:, 0:128] / den,
                 preferred_element_type=jnp.float32)
    g2 = jnp.dot(gmat, sums[:, 128:256] / den,
                 preferred_element_type=jnp.float32)
    g3 = jnp.dot(gmat, sums[:, 256:384] / den,
                 preferred_element_type=jnp.float32)
    gcat = jnp.concatenate([g1, g2, g3], axis=1)
    hh = jnp.maximum(
        jnp.dot(gcat, w1_ref[...], preferred_element_type=jnp.float32)
        + b1_ref[...], 0.0)
    o_ref[...] = (jnp.dot(hh, w2_ref[...], preferred_element_type=jnp.float32)
                  + b2_ref[...])


_sc_params = pltpu.CompilerParams(needs_layout_passes=False)

_k2_call = functools.partial(
    pl.kernel, _k2_mask, mesh=_mesh, compiler_params=_sc_params,
    out_type=[jax.ShapeDtypeStruct((NW, EPW), jnp.int32),
              jax.ShapeDtypeStruct((NW, NPAD), jnp.float32)],
    scratch_types=[pltpu.VMEM((NPAD,), jnp.float32),
                   pltpu.VMEM((16,), jnp.float32),
                   pltpu.VMEM((EPW,), jnp.int32),
                   pltpu.VMEM((EPW,), jnp.int32),
                   pltpu.VMEM((EPW,), jnp.int32),
                   pltpu.VMEM((NPAD,), jnp.float32)])

_k3_call = functools.partial(
    pl.kernel, _k3_msg, mesh=_mesh, compiler_params=_sc_params,
    out_type=jax.ShapeDtypeStruct((2, NPAD, D), jnp.float32),
    scratch_types=[pltpu.VMEM((CH, 128), jnp.int32),
                   pltpu.VMEM((NBUF, 128), jnp.int32),
                   pltpu.VMEM((NBUF, 128, D), jnp.float32),
                   pltpu.VMEM_SHARED((NPAD, D), jnp.float32),
                   pltpu.SemaphoreType.DMA((NBUF,)),
                   pltpu.SemaphoreType.DMA((NBUF,)),
                   pltpu.SemaphoreType.DMA((NBUF,))])


def kernel(x, edge_index, edge_attr, node_to_subgraph, edge_to_subgraph,
           subgraph_to_graph, params):
    f32 = jnp.float32
    src, dst = edge_index[0], edge_index[1]

    xp = jnp.zeros((NPAD, D), f32).at[:N].set(x)
    npadE = EPAD - E
    srcp = jnp.concatenate([src, jnp.full((npadE,), N, jnp.int32)])
    dstp = jnp.concatenate([dst, (jnp.arange(npadE, dtype=jnp.int32) % 64)])
    batchp = jnp.concatenate(
        [node_to_subgraph, jnp.full((NPAD - N,), SUBPAD - 1, jnp.int32)])
    s2gp = jnp.concatenate(
        [subgraph_to_graph,
         jnp.full((SUBPAD - NUM_SUB,), NUM_GRAPHS, jnp.int32)])

    xw79, tvec = pl.pallas_call(
        _k1_score,
        out_shape=[jax.ShapeDtypeStruct((NBLK, 128), f32),
                   jax.ShapeDtypeStruct((1, 128), f32)],
    )(xp.reshape(NBLK, 128, D), params['pool_w'].reshape(1, D))
    xw = xw79.reshape(NPAD)
    t16 = tvec.reshape(128)[:16]

    dstm, touch = _k2_call()(xw, t16, srcp, dstp)
    src3 = srcp.reshape(NW, CH, 128)
    dst3 = dstm.reshape(NW, CH, 128)

    zeros_tile = jnp.zeros((ROWS_PER_TILE, D), f32)

    nblk = 8
    rblk = NPAD // nblk
    h = xp
    hs = []
    for l in range(N_LAYERS):
        msg2 = _k3_call()(h, src3, dst3, zeros_tile)
        epsv = jnp.full((1, HID), 1.0 + params['eps_%d' % l], f32)
        h = pl.pallas_call(
            _k4_mlp,
            grid=(nblk,),
            in_specs=[
                pl.BlockSpec((rblk, D), lambda i: (i, 0)),
                pl.BlockSpec((rblk, D), lambda i: (i, 0)),
                pl.BlockSpec((rblk, D), lambda i: (i, 0)),
                pl.BlockSpec((1, HID), lambda i: (0, 0)),
                pl.BlockSpec((D, HID), lambda i: (0, 0)),
                pl.BlockSpec((1, HID), lambda i: (0, 0)),
                pl.BlockSpec((HID, HID), lambda i: (0, 0)),
                pl.BlockSpec((1, HID), lambda i: (0, 0)),
            ],
            out_specs=pl.BlockSpec((rblk, D), lambda i: (i, 0)),
            out_shape=jax.ShapeDtypeStruct((NPAD, HID), f32),
        )(h, msg2[0], msg2[1], epsv, params['W1_%d' % l],
          params['b1_%d' % l].reshape(1, HID), params['W2_%d' % l],
          params['b2_%d' % l].reshape(1, HID))
        hs.append(h)

    sums = pl.pallas_call(
        _k5_pool,
        grid=(NBLK,),
        in_specs=[
            pl.BlockSpec((1, 1, 128), lambda i: (i, 0, 0)),
            pl.BlockSpec((NW, 128), lambda i: (0, i)),
            pl.BlockSpec((128, HID), lambda i: (i, 0)),
            pl.BlockSpec((128, HID), lambda i: (i, 0)),
            pl.BlockSpec((128, HID), lambda i: (i, 0)),
        ],
        out_specs=pl.BlockSpec((SUBPAD, FEAT), lambda i: (0, 0)),
        out_shape=jax.ShapeDtypeStruct((SUBPAD, FEAT), f32),
        scratch_shapes=[pltpu.VMEM((128, FEAT), jnp.bfloat16)],
    )(batchp.reshape(NBLK, 1, 128), touch, hs[0], hs[1], hs[2])

    w2p = jnp.zeros((HID, 128), f32).at[:, :OUT_DIM].set(params['lin2_W'])
    b2p = jnp.zeros((128,), f32).at[:OUT_DIM].set(params['lin2_b'])
    outp = pl.pallas_call(
        _k6_final,
        out_shape=jax.ShapeDtypeStruct((NUM_GRAPHS, 128), f32),
    )(sums, s2gp.reshape(8, 128), params['lin1_W'],
      params['lin1_b'].reshape(1, HID), w2p, b2p.reshape(1, 128))
    return outp[:, :OUT_DIM]


# speedup vs baseline: 1.0235x; 1.0235x over previous
import functools

import jax
import jax.numpy as jnp
import numpy as np
from jax import lax
from jax.experimental import pallas as pl
from jax.experimental.pallas import tpu as pltpu
from jax.experimental.pallas import tpu_sc as plsc

N = 10000
E = 320000
D = 128
HID = 128
OUT_DIM = 64
N_LAYERS = 3
NUM_SUB = 1000
NUM_GRAPHS = 16

NPAD = 10112
NBLK = NPAD // 128
NW = 32
EPW = 10240
EPAD = NW * EPW
CH = EPW // 128
NBUF = 2
NIBUF = 4
ROWS_PER_TILE = NPAD // 16
LOGIT_MIN_SCORE = float(np.log(0.3 / 0.7))
SUBPAD = 1024
FEAT = 512

_mesh = plsc.VectorSubcoreMesh(core_axis_name="c", subcore_axis_name="s")


def _k1_score(x3_ref, w_ref, xw_ref, t_ref):
    x3 = x3_ref[...]
    w = w_ref[...]
    xw = jnp.sum(x3 * w[None, :, :], axis=2)
    flat = (lax.broadcasted_iota(jnp.int32, (NBLK, 128), 0) * 128
            + lax.broadcasted_iota(jnp.int32, (NBLK, 128), 1))
    xw_ref[...] = jnp.where(flat < N, xw, -1e30)
    nrm = jnp.sqrt(jnp.sum(w * w)) + 1e-12
    t_ref[...] = jnp.full((1, 128), LOGIT_MIN_SCORE, jnp.float32) * nrm


def _k2_mask(xw_hbm, t_hbm, src_hbm, dst_hbm, dstm_out, touch_out,
             xw_v, t_v, src_v, dst_v, dstm_v, touch_v):
    c = lax.axis_index("c")
    s = lax.axis_index("s")
    w = c * 16 + s
    base = w * EPW
    pltpu.sync_copy(xw_hbm, xw_v)
    pltpu.sync_copy(t_hbm, t_v)
    pltpu.sync_copy(src_hbm.at[pl.ds(base, EPW)], src_v)
    pltpu.sync_copy(dst_hbm.at[pl.ds(base, EPW)], dst_v)
    t16 = t_v[...]
    ones = jnp.full((16,), 1.0, jnp.float32)

    def zbody(i, carry):
        touch_v[pl.ds(i * 16, 16)] = jnp.zeros((16,), jnp.float32)
        return carry

    lax.fori_loop(0, NPAD // 16, zbody, 0)

    def body(i, carry):
        sl = pl.ds(i * 16, 16)
        s16 = src_v[sl]
        d16 = dst_v[sl]
        z = plsc.load_gather(xw_v, [s16]) + plsc.load_gather(xw_v, [d16])
        keep = z > t16
        dstm_v[sl] = jnp.where(keep, d16, N + (d16 & 63))
        plsc.store_scatter(touch_v, [s16], ones, mask=keep)
        plsc.store_scatter(touch_v, [d16], ones, mask=keep)
        return carry

    lax.fori_loop(0, EPW // 16, body, 0)
    pltpu.sync_copy(dstm_v, dstm_out.at[w])
    pltpu.sync_copy(touch_v, touch_out.at[w])


def _k3_msg(h_hbm, src3_hbm, dst3_hbm, zeros_hbm, out_hbm,
            sidx_v, didx_v, stage_v, acc_sh, gsem, isem):
    c = lax.axis_index("c")
    s = lax.axis_index("s")
    w = c * 16 + s
    rows = pl.ds(s * ROWS_PER_TILE, ROWS_PER_TILE)
    pltpu.sync_copy(zeros_hbm, acc_sh.at[rows])
    pltpu.sync_copy(src3_hbm.at[w], sidx_v)
    plsc.subcore_barrier()

    for b in range(NBUF):
        pltpu.async_copy(h_hbm.at[sidx_v.at[b]], stage_v.at[b], gsem.at[b])
    for r in range(NIBUF):
        pltpu.async_copy(dst3_hbm.at[w].at[r], didx_v.at[r], isem.at[r])

    def body(t, carry):
        for b in range(NBUF):
            cc = t * NBUF + b
            r = (t * NBUF + b) % NIBUF
            pltpu.make_async_copy(
                h_hbm.at[sidx_v.at[cc]], stage_v.at[b], gsem.at[b]).wait()
            pltpu.make_async_copy(
                dst3_hbm.at[w].at[cc], didx_v.at[r], isem.at[r]).wait()
            pltpu.sync_copy(stage_v.at[b], acc_sh.at[didx_v.at[r]], add=True)
            cn = cc + NBUF
            rn = cc + NIBUF

            @pl.when(cn < CH)
            def _():
                pltpu.async_copy(h_hbm.at[sidx_v.at[cn]], stage_v.at[b],
                                 gsem.at[b])

            @pl.when(rn < CH)
            def _():
                pltpu.async_copy(dst3_hbm.at[w].at[rn], didx_v.at[r],
                                 isem.at[r])
        return carry

    lax.fori_loop(0, CH // NBUF, body, 0)
    plsc.subcore_barrier()
    pltpu.sync_copy(acc_sh.at[rows], out_hbm.at[c].at[rows])


def _k4_mlp(h_ref, m0_ref, m1_ref, eps_ref, w1_ref, b1_ref, w2_ref, b2_ref,
            o_ref):
    agg = h_ref[...] * eps_ref[...] + m0_ref[...] + m1_ref[...]
    hh = jnp.maximum(
        jnp.dot(agg, w1_ref[...], preferred_element_type=jnp.float32)
        + b1_ref[...], 0.0)
    o_ref[...] = (jnp.dot(hh, w2_ref[...], preferred_element_type=jnp.float32)
                  + b2_ref[...])


def _k5_pool(b3_ref, t_ref, h1_ref, h2_ref, h3_ref, o_ref, x_scr):
    i = pl.program_id(0)
    batch = b3_ref[...].reshape(1, 128)
    m = (jnp.sum(t_ref[...], axis=0, keepdims=True) > 0.0).astype(
        jnp.bfloat16)
    seg = lax.broadcasted_iota(jnp.int32, (SUBPAD, 128), 0)
    sm = ((seg == batch).astype(jnp.float32).astype(jnp.bfloat16)
          * m)
    x_scr[:, 0:128] = h1_ref[...].astype(jnp.bfloat16)
    x_scr[:, 128:256] = h2_ref[...].astype(jnp.bfloat16)
    x_scr[:, 256:384] = h3_ref[...].astype(jnp.bfloat16)
    lane = lax.broadcasted_iota(jnp.int32, (128, 128), 1)
    x_scr[:, 384:512] = (lane == 0).astype(jnp.float32).astype(jnp.bfloat16)

    @pl.when(i == 0)
    def _():
        o_ref[...] = jnp.zeros((SUBPAD, FEAT), jnp.float32)

    o_ref[...] += jnp.dot(sm, x_scr[...],
                          preferred_element_type=jnp.float32)


def _k6_final(sums_ref, s2g_ref, w1_ref, b1_ref, w2_ref, b2_ref, o_ref):
    sums = sums_ref[...]
    s2g = s2g_ref[...].reshape(1, SUBPAD)
    cnt0 = sums[:, 384:512]
    ones_j = jnp.ones((128, 128), jnp.float32)
    den = jnp.maximum(
        jnp.dot(cnt0, ones_j, preferred_element_type=jnp.float32), 1.0)
    gmat = (lax.broadcasted_iota(jnp.int32, (NUM_GRAPHS, SUBPAD), 0)
            == s2g).astype(jnp.float32)
    g1 = jnp.dot(gmat, sums[:, 0:128] / den,
                 preferred_element_type=jnp.float32)
    g2 = jnp.dot(gmat, sums[:, 128:256] / den,
                 preferred_element_type=jnp.float32)
    g3 = jnp.dot(gmat, sums[:, 256:384] / den,
                 preferred_element_type=jnp.float32)
    gcat = jnp.concatenate([g1, g2, g3], axis=1)
    hh = jnp.maximum(
        jnp.dot(gcat, w1_ref[...], preferred_element_type=jnp.float32)
        + b1_ref[...], 0.0)
    o_ref[...] = (jnp.dot(hh, w2_ref[...], preferred_element_type=jnp.float32)
                  + b2_ref[...])


_sc_params = pltpu.CompilerParams(needs_layout_passes=False)

_k2_call = functools.partial(
    pl.kernel, _k2_mask, mesh=_mesh, compiler_params=_sc_params,
    out_type=[jax.ShapeDtypeStruct((NW, EPW), jnp.int32),
              jax.ShapeDtypeStruct((NW, NPAD), jnp.float32)],
    scratch_types=[pltpu.VMEM((NPAD,), jnp.float32),
                   pltpu.VMEM((16,), jnp.float32),
                   pltpu.VMEM((EPW,), jnp.int32),
                   pltpu.VMEM((EPW,), jnp.int32),
                   pltpu.VMEM((EPW,), jnp.int32),
                   pltpu.VMEM((NPAD,), jnp.float32)])

_k3_call = functools.partial(
    pl.kernel, _k3_msg, mesh=_mesh, compiler_params=_sc_params,
    out_type=jax.ShapeDtypeStruct((2, NPAD, D), jnp.float32),
    scratch_types=[pltpu.VMEM((CH, 128), jnp.int32),
                   pltpu.VMEM((NIBUF, 128), jnp.int32),
                   pltpu.VMEM((NBUF, 128, D), jnp.float32),
                   pltpu.VMEM_SHARED((NPAD, D), jnp.float32),
                   pltpu.SemaphoreType.DMA((NBUF,)),
                   pltpu.SemaphoreType.DMA((NIBUF,))])


def kernel(x, edge_index, edge_attr, node_to_subgraph, edge_to_subgraph,
           subgraph_to_graph, params):
    f32 = jnp.float32
    src, dst = edge_index[0], edge_index[1]

    xp = jnp.zeros((NPAD, D), f32).at[:N].set(x)
    npadE = EPAD - E
    srcp = jnp.concatenate([src, jnp.full((npadE,), N, jnp.int32)])
    dstp = jnp.concatenate([dst, (jnp.arange(npadE, dtype=jnp.int32) % 64)])
    batchp = jnp.concatenate(
        [node_to_subgraph, jnp.full((NPAD - N,), SUBPAD - 1, jnp.int32)])
    s2gp = jnp.concatenate(
        [subgraph_to_graph,
         jnp.full((SUBPAD - NUM_SUB,), NUM_GRAPHS, jnp.int32)])

    xw79, tvec = pl.pallas_call(
        _k1_score,
        out_shape=[jax.ShapeDtypeStruct((NBLK, 128), f32),
                   jax.ShapeDtypeStruct((1, 128), f32)],
    )(xp.reshape(NBLK, 128, D), params['pool_w'].reshape(1, D))
    xw = xw79.reshape(NPAD)
    t16 = tvec.reshape(128)[:16]

    dstm, touch = _k2_call()(xw, t16, srcp, dstp)
    src3 = srcp.reshape(NW, CH, 128)
    dst3 = dstm.reshape(NW, CH, 128)

    zeros_tile = jnp.zeros((ROWS_PER_TILE, D), f32)

    nblk = 8
    rblk = NPAD // nblk
    h = xp
    hs = []
    for l in range(N_LAYERS):
        msg2 = _k3_call()(h, src3, dst3, zeros_tile)
        epsv = jnp.full((1, HID), 1.0 + params['eps_%d' % l], f32)
        h = pl.pallas_call(
            _k4_mlp,
            grid=(nblk,),
            in_specs=[
                pl.BlockSpec((rblk, D), lambda i: (i, 0)),
                pl.BlockSpec((rblk, D), lambda i: (i, 0)),
                pl.BlockSpec((rblk, D), lambda i: (i, 0)),
                pl.BlockSpec((1, HID), lambda i: (0, 0)),
                pl.BlockSpec((D, HID), lambda i: (0, 0)),
                pl.BlockSpec((1, HID), lambda i: (0, 0)),
                pl.BlockSpec((HID, HID), lambda i: (0, 0)),
                pl.BlockSpec((1, HID), lambda i: (0, 0)),
            ],
            out_specs=pl.BlockSpec((rblk, D), lambda i: (i, 0)),
            out_shape=jax.ShapeDtypeStruct((NPAD, HID), f32),
        )(h, msg2[0], msg2[1], epsv, params['W1_%d' % l],
          params['b1_%d' % l].reshape(1, HID), params['W2_%d' % l],
          params['b2_%d' % l].reshape(1, HID))
        hs.append(h)

    sums = pl.pallas_call(
        _k5_pool,
        grid=(NBLK,),
        in_specs=[
            pl.BlockSpec((1, 1, 128), lambda i: (i, 0, 0)),
            pl.BlockSpec((NW, 128), lambda i: (0, i)),
            pl.BlockSpec((128, HID), lambda i: (i, 0)),
            pl.BlockSpec((128, HID), lambda i: (i, 0)),
            pl.BlockSpec((128, HID), lambda i: (i, 0)),
        ],
        out_specs=pl.BlockSpec((SUBPAD, FEAT), lambda i: (0, 0)),
        out_shape=jax.ShapeDtypeStruct((SUBPAD, FEAT), f32),
        scratch_shapes=[pltpu.VMEM((128, FEAT), jnp.bfloat16)],
    )(batchp.reshape(NBLK, 1, 128), touch, hs[0], hs[1], hs[2])

    w2p = jnp.zeros((HID, 128), f32).at[:, :OUT_DIM].set(params['lin2_W'])
    b2p = jnp.zeros((128,), f32).at[:OUT_DIM].set(params['lin2_b'])
    outp = pl.pallas_call(
        _k6_final,
        out_shape=jax.ShapeDtypeStruct((NUM_GRAPHS, 128), f32),
    )(sums, s2gp.reshape(8, 128), params['lin1_W'],
      params['lin1_b'].reshape(1, HID), w2p, b2p.reshape(1, 128))
    return outp[:, :OUT_DIM]


# speedup vs baseline: 1.9847x; 1.9392x over previous
import functools

import jax
import jax.numpy as jnp
import numpy as np
from jax import lax
from jax.experimental import pallas as pl
from jax.experimental.pallas import tpu as pltpu
from jax.experimental.pallas import tpu_sc as plsc

N = 10000
E = 320000
D = 128
HID = 128
OUT_DIM = 64
N_LAYERS = 3
NUM_SUB = 1000
NUM_GRAPHS = 16

NPAD = 10112
NBLK = NPAD // 128
NW = 32
EPW = 10240
EPAD = NW * EPW
CH = EPW // 128
EPB = 10496
CHB = EPB // 128
ROWS_PER_TILE = NPAD // 16
LOGIT_MIN_SCORE = float(np.log(0.3 / 0.7))
SUBPAD = 1024
FEAT = 512

_mesh = plsc.VectorSubcoreMesh(core_axis_name="c", subcore_axis_name="s")


def _k1_score(x3_ref, w_ref, xw_ref, t_ref):
    x3 = x3_ref[...]
    w = w_ref[...]
    xw = jnp.sum(x3 * w[None, :, :], axis=2)
    flat = (lax.broadcasted_iota(jnp.int32, (NBLK, 128), 0) * 128
            + lax.broadcasted_iota(jnp.int32, (NBLK, 128), 1))
    xw_ref[...] = jnp.where(flat < N, xw, -1e30)
    nrm = jnp.sqrt(jnp.sum(w * w)) + 1e-12
    t_ref[...] = jnp.full((1, 128), LOGIT_MIN_SCORE, jnp.float32) * nrm


def _k2_mask(xw_hbm, t_hbm, src_hbm, dst_hbm,
             srcc_out, dstc_out, cnt_out, touch_out,
             xw_v, t_v, src_v, dst_v, srcc_v, dstc_v, touch_v, cnt_v):
    c = lax.axis_index("c")
    s = lax.axis_index("s")
    w = c * 16 + s
    base = w * EPW
    pltpu.sync_copy(xw_hbm, xw_v)
    pltpu.sync_copy(t_hbm, t_v)
    pltpu.sync_copy(src_hbm.at[pl.ds(base, EPW)], src_v)
    pltpu.sync_copy(dst_hbm.at[pl.ds(base, EPW)], dst_v)
    t16 = t_v[...]
    ones = jnp.full((16,), 1.0, jnp.float32)
    padv = N + 64 + lax.broadcasted_iota(jnp.int32, (16,), 0)

    def zbody(i, carry):
        touch_v[pl.ds(i * 16, 16)] = jnp.zeros((16,), jnp.float32)
        return carry

    lax.fori_loop(0, NPAD // 16, zbody, 0)

    def body(i, cnt):
        sl = pl.ds(i * 16, 16)
        s16 = src_v[sl]
        d16 = dst_v[sl]
        z = plsc.load_gather(xw_v, [s16]) + plsc.load_gather(xw_v, [d16])
        keep = z > t16
        plsc.store_scatter(touch_v, [s16], ones, mask=keep)
        plsc.store_scatter(touch_v, [d16], ones, mask=keep)
        plsc.store_compressed(srcc_v.at[pl.ds(cnt, 16)], s16, mask=keep)
        plsc.store_compressed(dstc_v.at[pl.ds(cnt, 16)], d16, mask=keep)
        return cnt + plsc.all_reduce_population_count(keep)[0]

    cnt = lax.fori_loop(0, EPW // 16, body, jnp.int32(0))
    for q in range(8):
        srcc_v[pl.ds(cnt + q * 16, 16)] = jnp.zeros((16,), jnp.int32)
        dstc_v[pl.ds(cnt + q * 16, 16)] = padv
    nch = (cnt + 127) // 128
    cnt_v[...] = jnp.broadcast_to(nch, (16,))
    pltpu.sync_copy(srcc_v, srcc_out.at[w])
    pltpu.sync_copy(dstc_v, dstc_out.at[w])
    pltpu.sync_copy(cnt_v, cnt_out.at[w])
    pltpu.sync_copy(touch_v, touch_out.at[w])


def _k3_msg(h_hbm, src3_hbm, dst3_hbm, cnt_hbm, zeros_hbm, out_hbm,
            sidx_v, didx_v, stage_v, cnt_v, acc_sh, sem):
    c = lax.axis_index("c")
    s = lax.axis_index("s")
    w = c * 16 + s
    rows = pl.ds(s * ROWS_PER_TILE, ROWS_PER_TILE)
    pltpu.sync_copy(zeros_hbm, acc_sh.at[rows])
    pltpu.sync_copy(src3_hbm.at[w], sidx_v)
    pltpu.sync_copy(dst3_hbm.at[w], didx_v)
    pltpu.sync_copy(cnt_hbm.at[w], cnt_v)
    nch = cnt_v[...][0]
    plsc.subcore_barrier()

    def body(j, carry):
        pltpu.async_copy(h_hbm.at[sidx_v.at[j]], stage_v, sem).wait()
        pltpu.sync_copy(stage_v, acc_sh.at[didx_v.at[j]], add=True)
        return carry

    lax.fori_loop(0, nch, body, 0)
    plsc.subcore_barrier()
    pltpu.sync_copy(acc_sh.at[rows], out_hbm.at[c].at[rows])


def _k4_mlp(h_ref, m0_ref, m1_ref, eps_ref, w1_ref, b1_ref, w2_ref, b2_ref,
            o_ref):
    agg = h_ref[...] * eps_ref[...] + m0_ref[...] + m1_ref[...]
    hh = jnp.maximum(
        jnp.dot(agg, w1_ref[...], preferred_element_type=jnp.float32)
        + b1_ref[...], 0.0)
    o_ref[...] = (jnp.dot(hh, w2_ref[...], preferred_element_type=jnp.float32)
                  + b2_ref[...])


def _k5_pool(b3_ref, t_ref, h1_ref, h2_ref, h3_ref, o_ref, x_scr):
    i = pl.program_id(0)
    batch = b3_ref[...].reshape(1, 128)
    m = (jnp.sum(t_ref[...], axis=0, keepdims=True) > 0.0).astype(
        jnp.bfloat16)
    seg = lax.broadcasted_iota(jnp.int32, (SUBPAD, 128), 0)
    sm = ((seg == batch).astype(jnp.float32).astype(jnp.bfloat16)
          * m)
    x_scr[:, 0:128] = h1_ref[...].astype(jnp.bfloat16)
    x_scr[:, 128:256] = h2_ref[...].astype(jnp.bfloat16)
    x_scr[:, 256:384] = h3_ref[...].astype(jnp.bfloat16)
    lane = lax.broadcasted_iota(jnp.int32, (128, 128), 1)
    x_scr[:, 384:512] = (lane == 0).astype(jnp.float32).astype(jnp.bfloat16)

    @pl.when(i == 0)
    def _():
        o_ref[...] = jnp.zeros((SUBPAD, FEAT), jnp.float32)

    o_ref[...] += jnp.dot(sm, x_scr[...],
                          preferred_element_type=jnp.float32)


def _k6_final(sums_ref, s2g_ref, w1_ref, b1_ref, w2_ref, b2_ref, o_ref):
    sums = sums_ref[...]
    s2g = s2g_ref[...].reshape(1, SUBPAD)
    cnt0 = sums[:, 384:512]
    ones_j = jnp.ones((128, 128), jnp.float32)
    den = jnp.maximum(
        jnp.dot(cnt0, ones_j, preferred_element_type=jnp.float32), 1.0)
    gmat = (lax.broadcasted_iota(jnp.int32, (NUM_GRAPHS, SUBPAD), 0)
            == s2g).astype(jnp.float32)
    g1 = jnp.dot(gmat, sums[:, 0:128] / den,
                 preferred_element_type=jnp.float32)
    g2 = jnp.dot(gmat, sums[:, 128:256] / den,
                 preferred_element_type=jnp.float32)
    g3 = jnp.dot(gmat, sums[:, 256:384] / den,
                 preferred_element_type=jnp.float32)
    gcat = jnp.concatenate([g1, g2, g3], axis=1)
    hh = jnp.maximum(
        jnp.dot(gcat, w1_ref[...], preferred_element_type=jnp.float32)
        + b1_ref[...], 0.0)
    o_ref[...] = (jnp.dot(hh, w2_ref[...], preferred_element_type=jnp.float32)
                  + b2_ref[...])


_sc_params = pltpu.CompilerParams(needs_layout_passes=False)

_k2_call = functools.partial(
    pl.kernel, _k2_mask, mesh=_mesh, compiler_params=_sc_params,
    out_type=[jax.ShapeDtypeStruct((NW, EPB), jnp.int32),
              jax.ShapeDtypeStruct((NW, EPB), jnp.int32),
              jax.ShapeDtypeStruct((NW, 16), jnp.int32),
              jax.ShapeDtypeStruct((NW, NPAD), jnp.float32)],
    scratch_types=[pltpu.VMEM((NPAD,), jnp.float32),
                   pltpu.VMEM((16,), jnp.float32),
                   pltpu.VMEM((EPW,), jnp.int32),
                   pltpu.VMEM((EPW,), jnp.int32),
                   pltpu.VMEM((EPB,), jnp.int32),
                   pltpu.VMEM((EPB,), jnp.int32),
                   pltpu.VMEM((NPAD,), jnp.float32),
                   pltpu.VMEM((16,), jnp.int32)])

_k3_call = functools.partial(
    pl.kernel, _k3_msg, mesh=_mesh, compiler_params=_sc_params,
    out_type=jax.ShapeDtypeStruct((2, NPAD, D), jnp.float32),
    scratch_types=[pltpu.VMEM((CHB, 128), jnp.int32),
                   pltpu.VMEM((CHB, 128), jnp.int32),
                   pltpu.VMEM((128, D), jnp.float32),
                   pltpu.VMEM((16,), jnp.int32),
                   pltpu.VMEM_SHARED((NPAD, D), jnp.float32),
                   pltpu.SemaphoreType.DMA])


def kernel(x, edge_index, edge_attr, node_to_subgraph, edge_to_subgraph,
           subgraph_to_graph, params):
    f32 = jnp.float32
    src, dst = edge_index[0], edge_index[1]

    xp = jnp.zeros((NPAD, D), f32).at[:N].set(x)
    npadE = EPAD - E
    srcp = jnp.concatenate([src, jnp.full((npadE,), N, jnp.int32)])
    dstp = jnp.concatenate([dst, (jnp.arange(npadE, dtype=jnp.int32) % 64)])
    batchp = jnp.concatenate(
        [node_to_subgraph, jnp.full((NPAD - N,), SUBPAD - 1, jnp.int32)])
    s2gp = jnp.concatenate(
        [subgraph_to_graph,
         jnp.full((SUBPAD - NUM_SUB,), NUM_GRAPHS, jnp.int32)])

    xw79, tvec = pl.pallas_call(
        _k1_score,
        out_shape=[jax.ShapeDtypeStruct((NBLK, 128), f32),
                   jax.ShapeDtypeStruct((1, 128), f32)],
    )(xp.reshape(NBLK, 128, D), params['pool_w'].reshape(1, D))
    xw = xw79.reshape(NPAD)
    t16 = tvec.reshape(128)[:16]

    srcc, dstc, cnts, touch = _k2_call()(xw, t16, srcp, dstp)
    src3 = srcc.reshape(NW, CHB, 128)
    dst3 = dstc.reshape(NW, CHB, 128)

    zeros_tile = jnp.zeros((ROWS_PER_TILE, D), f32)

    nblk = 8
    rblk = NPAD // nblk
    h = xp
    hs = []
    for l in range(N_LAYERS):
        msg2 = _k3_call()(h, src3, dst3, cnts, zeros_tile)
        epsv = jnp.full((1, HID), 1.0 + params['eps_%d' % l], f32)
        h = pl.pallas_call(
            _k4_mlp,
            grid=(nblk,),
            in_specs=[
                pl.BlockSpec((rblk, D), lambda i: (i, 0)),
                pl.BlockSpec((rblk, D), lambda i: (i, 0)),
                pl.BlockSpec((rblk, D), lambda i: (i, 0)),
                pl.BlockSpec((1, HID), lambda i: (0, 0)),
                pl.BlockSpec((D, HID), lambda i: (0, 0)),
                pl.BlockSpec((1, HID), lambda i: (0, 0)),
                pl.BlockSpec((HID, HID), lambda i: (0, 0)),
                pl.BlockSpec((1, HID), lambda i: (0, 0)),
            ],
            out_specs=pl.BlockSpec((rblk, D), lambda i: (i, 0)),
            out_shape=jax.ShapeDtypeStruct((NPAD, HID), f32),
        )(h, msg2[0], msg2[1], epsv, params['W1_%d' % l],
          params['b1_%d' % l].reshape(1, HID), params['W2_%d' % l],
          params['b2_%d' % l].reshape(1, HID))
        hs.append(h)

    sums = pl.pallas_call(
        _k5_pool,
        grid=(NBLK,),
        in_specs=[
            pl.BlockSpec((1, 1, 128), lambda i: (i, 0, 0)),
            pl.BlockSpec((NW, 128), lambda i: (0, i)),
            pl.BlockSpec((128, HID), lambda i: (i, 0)),
            pl.BlockSpec((128, HID), lambda i: (i, 0)),
            pl.BlockSpec((128, HID), lambda i: (i, 0)),
        ],
        out_specs=pl.BlockSpec((SUBPAD, FEAT), lambda i: (0, 0)),
        out_shape=jax.ShapeDtypeStruct((SUBPAD, FEAT), f32),
        scratch_shapes=[pltpu.VMEM((128, FEAT), jnp.bfloat16)],
    )(batchp.reshape(NBLK, 1, 128), touch, hs[0], hs[1], hs[2])

    w2p = jnp.zeros((HID, 128), f32).at[:, :OUT_DIM].set(params['lin2_W'])
    b2p = jnp.zeros((128,), f32).at[:OUT_DIM].set(params['lin2_b'])
    outp = pl.pallas_call(
        _k6_final,
        out_shape=jax.ShapeDtypeStruct((NUM_GRAPHS, 128), f32),
    )(sums, s2gp.reshape(8, 128), params['lin1_W'],
      params['lin1_b'].reshape(1, HID), w2p, b2p.reshape(1, 128))
    return outp[:, :OUT_DIM]


# speedup vs baseline: 2.0126x; 1.0141x over previous
import functools

import jax
import jax.numpy as jnp
import numpy as np
from jax import lax
from jax.experimental import pallas as pl
from jax.experimental.pallas import tpu as pltpu
from jax.experimental.pallas import tpu_sc as plsc

N = 10000
E = 320000
D = 128
HID = 128
OUT_DIM = 64
N_LAYERS = 3
NUM_SUB = 1000
NUM_GRAPHS = 16

NPAD = 10112
NBLK = NPAD // 128
NW = 32
EPW = 10240
EPAD = NW * EPW
CH = EPW // 128
EPB = 10496
CHB = EPB // 128
ROWS_PER_TILE = NPAD // 16
LOGIT_MIN_SCORE = float(np.log(0.3 / 0.7))
SUBPAD = 1024
FEAT = 512

_mesh = plsc.VectorSubcoreMesh(core_axis_name="c", subcore_axis_name="s")


def _k1_score(x3_ref, w_ref, xw_ref, t_ref):
    x3 = x3_ref[...]
    w = w_ref[...]
    xw = jnp.sum(x3 * w[None, :, :], axis=2)
    flat = (lax.broadcasted_iota(jnp.int32, (NBLK, 128), 0) * 128
            + lax.broadcasted_iota(jnp.int32, (NBLK, 128), 1))
    xw_ref[...] = jnp.where(flat < N, xw, -1e30)
    nrm = jnp.sqrt(jnp.sum(w * w)) + 1e-12
    t_ref[...] = jnp.full((1, 128), LOGIT_MIN_SCORE, jnp.float32) * nrm


def _k2_mask(xw_hbm, t_hbm, src_hbm, dst_hbm,
             srcc_out, dstc_out, cnt_out, touch_out,
             xw_v, t_v, src_v, dst_v, srcc_v, dstc_v, touch_v, cnt_v):
    c = lax.axis_index("c")
    s = lax.axis_index("s")
    w = c * 16 + s
    base = w * EPW
    pltpu.sync_copy(xw_hbm, xw_v)
    pltpu.sync_copy(t_hbm, t_v)
    pltpu.sync_copy(src_hbm.at[pl.ds(base, EPW)], src_v)
    pltpu.sync_copy(dst_hbm.at[pl.ds(base, EPW)], dst_v)
    t16 = t_v[...]
    ones = jnp.full((16,), 1.0, jnp.float32)
    padv = N + 64 + lax.broadcasted_iota(jnp.int32, (16,), 0)

    def zbody(i, carry):
        touch_v[pl.ds(i * 16, 16)] = jnp.zeros((16,), jnp.float32)
        return carry

    lax.fori_loop(0, NPAD // 16, zbody, 0)

    def body(i, cnt):
        sl = pl.ds(i * 16, 16)
        s16 = src_v[sl]
        d16 = dst_v[sl]
        z = plsc.load_gather(xw_v, [s16]) + plsc.load_gather(xw_v, [d16])
        keep = z > t16
        plsc.store_scatter(touch_v, [s16], ones, mask=keep)
        plsc.store_scatter(touch_v, [d16], ones, mask=keep)
        plsc.store_compressed(srcc_v.at[pl.ds(cnt, 16)], s16, mask=keep)
        plsc.store_compressed(dstc_v.at[pl.ds(cnt, 16)], d16, mask=keep)
        return cnt + plsc.all_reduce_population_count(keep)[0]

    cnt = lax.fori_loop(0, EPW // 16, body, jnp.int32(0))
    for q in range(8):
        srcc_v[pl.ds(cnt + q * 16, 16)] = jnp.zeros((16,), jnp.int32)
        dstc_v[pl.ds(cnt + q * 16, 16)] = padv
    nch = (cnt + 127) // 128
    cnt_v[...] = jnp.broadcast_to(nch, (16,))
    pltpu.sync_copy(srcc_v, srcc_out.at[w])
    pltpu.sync_copy(dstc_v, dstc_out.at[w])
    pltpu.sync_copy(cnt_v, cnt_out.at[w])
    pltpu.sync_copy(touch_v, touch_out.at[w])


def _k3_msg(h_hbm, src3_hbm, dst3_hbm, cnt_hbm, zeros_hbm, out_hbm,
            sidx_v, didx_v, stage_v, cnt_v, acc_sh, sema, semb):
    c = lax.axis_index("c")
    s = lax.axis_index("s")
    w = c * 16 + s
    rows = pl.ds(s * ROWS_PER_TILE, ROWS_PER_TILE)
    pltpu.sync_copy(zeros_hbm, acc_sh.at[rows])
    pltpu.sync_copy(src3_hbm.at[w], sidx_v)
    pltpu.sync_copy(dst3_hbm.at[w], didx_v)
    pltpu.sync_copy(cnt_hbm.at[w], cnt_v)
    nch = cnt_v[...][0]
    plsc.subcore_barrier()

    def body(j, carry):
        da = pltpu.async_copy(h_hbm.at[sidx_v.at[j, pl.ds(0, 64)]],
                              stage_v.at[pl.ds(0, 64)], sema)
        db = pltpu.async_copy(h_hbm.at[sidx_v.at[j, pl.ds(64, 64)]],
                              stage_v.at[pl.ds(64, 64)], semb)
        da.wait()
        db.wait()
        pltpu.sync_copy(stage_v, acc_sh.at[didx_v.at[j]], add=True)
        return carry

    lax.fori_loop(0, nch, body, 0)
    plsc.subcore_barrier()
    pltpu.sync_copy(acc_sh.at[rows], out_hbm.at[c].at[rows])


def _k4_mlp(h_ref, m0_ref, m1_ref, eps_ref, w1_ref, b1_ref, w2_ref, b2_ref,
            o_ref):
    agg = h_ref[...] * eps_ref[...] + m0_ref[...] + m1_ref[...]
    hh = jnp.maximum(
        jnp.dot(agg, w1_ref[...], preferred_element_type=jnp.float32)
        + b1_ref[...], 0.0)
    o_ref[...] = (jnp.dot(hh, w2_ref[...], preferred_element_type=jnp.float32)
                  + b2_ref[...])


def _k5_pool(b3_ref, t_ref, h1_ref, h2_ref, h3_ref, o_ref, x_scr):
    i = pl.program_id(0)
    batch = b3_ref[...].reshape(1, 128)
    m = (jnp.sum(t_ref[...], axis=0, keepdims=True) > 0.0).astype(
        jnp.bfloat16)
    seg = lax.broadcasted_iota(jnp.int32, (SUBPAD, 128), 0)
    sm = ((seg == batch).astype(jnp.float32).astype(jnp.bfloat16)
          * m)
    x_scr[:, 0:128] = h1_ref[...].astype(jnp.bfloat16)
    x_scr[:, 128:256] = h2_ref[...].astype(jnp.bfloat16)
    x_scr[:, 256:384] = h3_ref[...].astype(jnp.bfloat16)
    lane = lax.broadcasted_iota(jnp.int32, (128, 128), 1)
    x_scr[:, 384:512] = (lane == 0).astype(jnp.float32).astype(jnp.bfloat16)

    @pl.when(i == 0)
    def _():
        o_ref[...] = jnp.zeros((SUBPAD, FEAT), jnp.float32)

    o_ref[...] += jnp.dot(sm, x_scr[...],
                          preferred_element_type=jnp.float32)


def _k6_final(sums_ref, s2g_ref, w1_ref, b1_ref, w2_ref, b2_ref, o_ref):
    sums = sums_ref[...]
    s2g = s2g_ref[...].reshape(1, SUBPAD)
    cnt0 = sums[:, 384:512]
    ones_j = jnp.ones((128, 128), jnp.float32)
    den = jnp.maximum(
        jnp.dot(cnt0, ones_j, preferred_element_type=jnp.float32), 1.0)
    gmat = (lax.broadcasted_iota(jnp.int32, (NUM_GRAPHS, SUBPAD), 0)
            == s2g).astype(jnp.float32)
    g1 = jnp.dot(gmat, sums[:, 0:128] / den,
                 preferred_element_type=jnp.float32)
    g2 = jnp.dot(gmat, sums[:, 128:256] / den,
                 preferred_element_type=jnp.float32)
    g3 = jnp.dot(gmat, sums[:, 256:384] / den,
                 preferred_element_type=jnp.float32)
    gcat = jnp.concatenate([g1, g2, g3], axis=1)
    hh = jnp.maximum(
        jnp.dot(gcat, w1_ref[...], preferred_element_type=jnp.float32)
        + b1_ref[...], 0.0)
    o_ref[...] = (jnp.dot(hh, w2_ref[...], preferred_element_type=jnp.float32)
                  + b2_ref[...])


_sc_params = pltpu.CompilerParams(needs_layout_passes=False)

_k2_call = functools.partial(
    pl.kernel, _k2_mask, mesh=_mesh, compiler_params=_sc_params,
    out_type=[jax.ShapeDtypeStruct((NW, EPB), jnp.int32),
              jax.ShapeDtypeStruct((NW, EPB), jnp.int32),
              jax.ShapeDtypeStruct((NW, 16), jnp.int32),
              jax.ShapeDtypeStruct((NW, NPAD), jnp.float32)],
    scratch_types=[pltpu.VMEM((NPAD,), jnp.float32),
                   pltpu.VMEM((16,), jnp.float32),
                   pltpu.VMEM((EPW,), jnp.int32),
                   pltpu.VMEM((EPW,), jnp.int32),
                   pltpu.VMEM((EPB,), jnp.int32),
                   pltpu.VMEM((EPB,), jnp.int32),
                   pltpu.VMEM((NPAD,), jnp.float32),
                   pltpu.VMEM((16,), jnp.int32)])

_k3_call = functools.partial(
    pl.kernel, _k3_msg, mesh=_mesh, compiler_params=_sc_params,
    out_type=jax.ShapeDtypeStruct((2, NPAD, D), jnp.float32),
    scratch_types=[pltpu.VMEM((CHB, 128), jnp.int32),
                   pltpu.VMEM((CHB, 128), jnp.int32),
                   pltpu.VMEM((128, D), jnp.float32),
                   pltpu.VMEM((16,), jnp.int32),
                   pltpu.VMEM_SHARED((NPAD, D), jnp.float32),
                   pltpu.SemaphoreType.DMA,
                   pltpu.SemaphoreType.DMA])


def kernel(x, edge_index, edge_attr, node_to_subgraph, edge_to_subgraph,
           subgraph_to_graph, params):
    f32 = jnp.float32
    src, dst = edge_index[0], edge_index[1]

    xp = jnp.zeros((NPAD, D), f32).at[:N].set(x)
    npadE = EPAD - E
    srcp = jnp.concatenate([src, jnp.full((npadE,), N, jnp.int32)])
    dstp = jnp.concatenate([dst, (jnp.arange(npadE, dtype=jnp.int32) % 64)])
    batchp = jnp.concatenate(
        [node_to_subgraph, jnp.full((NPAD - N,), SUBPAD - 1, jnp.int32)])
    s2gp = jnp.concatenate(
        [subgraph_to_graph,
         jnp.full((SUBPAD - NUM_SUB,), NUM_GRAPHS, jnp.int32)])

    xw79, tvec = pl.pallas_call(
        _k1_score,
        out_shape=[jax.ShapeDtypeStruct((NBLK, 128), f32),
                   jax.ShapeDtypeStruct((1, 128), f32)],
    )(xp.reshape(NBLK, 128, D), params['pool_w'].reshape(1, D))
    xw = xw79.reshape(NPAD)
    t16 = tvec.reshape(128)[:16]

    srcc, dstc, cnts, touch = _k2_call()(xw, t16, srcp, dstp)
    src3 = srcc.reshape(NW, CHB, 128)
    dst3 = dstc.reshape(NW, CHB, 128)

    zeros_tile = jnp.zeros((ROWS_PER_TILE, D), f32)

    nblk = 8
    rblk = NPAD // nblk
    h = xp
    hs = []
    for l in range(N_LAYERS):
        msg2 = _k3_call()(h, src3, dst3, cnts, zeros_tile)
        epsv = jnp.full((1, HID), 1.0 + params['eps_%d' % l], f32)
        h = pl.pallas_call(
            _k4_mlp,
            grid=(nblk,),
            in_specs=[
                pl.BlockSpec((rblk, D), lambda i: (i, 0)),
                pl.BlockSpec((rblk, D), lambda i: (i, 0)),
                pl.BlockSpec((rblk, D), lambda i: (i, 0)),
                pl.BlockSpec((1, HID), lambda i: (0, 0)),
                pl.BlockSpec((D, HID), lambda i: (0, 0)),
                pl.BlockSpec((1, HID), lambda i: (0, 0)),
                pl.BlockSpec((HID, HID), lambda i: (0, 0)),
                pl.BlockSpec((1, HID), lambda i: (0, 0)),
            ],
            out_specs=pl.BlockSpec((rblk, D), lambda i: (i, 0)),
            out_shape=jax.ShapeDtypeStruct((NPAD, HID), f32),
        )(h, msg2[0], msg2[1], epsv, params['W1_%d' % l],
          params['b1_%d' % l].reshape(1, HID), params['W2_%d' % l],
          params['b2_%d' % l].reshape(1, HID))
        hs.append(h)

    sums = pl.pallas_call(
        _k5_pool,
        grid=(NBLK,),
        in_specs=[
            pl.BlockSpec((1, 1, 128), lambda i: (i, 0, 0)),
            pl.BlockSpec((NW, 128), lambda i: (0, i)),
            pl.BlockSpec((128, HID), lambda i: (i, 0)),
            pl.BlockSpec((128, HID), lambda i: (i, 0)),
            pl.BlockSpec((128, HID), lambda i: (i, 0)),
        ],
        out_specs=pl.BlockSpec((SUBPAD, FEAT), lambda i: (0, 0)),
        out_shape=jax.ShapeDtypeStruct((SUBPAD, FEAT), f32),
        scratch_shapes=[pltpu.VMEM((128, FEAT), jnp.bfloat16)],
    )(batchp.reshape(NBLK, 1, 128), touch, hs[0], hs[1], hs[2])

    w2p = jnp.zeros((HID, 128), f32).at[:, :OUT_DIM].set(params['lin2_W'])
    b2p = jnp.zeros((128,), f32).at[:OUT_DIM].set(params['lin2_b'])
    outp = pl.pallas_call(
        _k6_final,
        out_shape=jax.ShapeDtypeStruct((NUM_GRAPHS, 128), f32),
    )(sums, s2gp.reshape(8, 128), params['lin1_W'],
      params['lin1_b'].reshape(1, HID), w2p, b2p.reshape(1, 128))
    return outp[:, :OUT_DIM]


# speedup vs baseline: 2.0177x; 1.0025x over previous
import functools

import jax
import jax.numpy as jnp
import numpy as np
from jax import lax
from jax.experimental import pallas as pl
from jax.experimental.pallas import tpu as pltpu
from jax.experimental.pallas import tpu_sc as plsc

N = 10000
E = 320000
D = 128
HID = 128
OUT_DIM = 64
N_LAYERS = 3
NUM_SUB = 1000
NUM_GRAPHS = 16

NPAD = 10112
NBLK = NPAD // 128
NW = 32
EPW = 10240
EPAD = NW * EPW
CH = EPW // 128
EPB = 10496
CHB = EPB // 128
ROWS_PER_TILE = NPAD // 16
LOGIT_MIN_SCORE = float(np.log(0.3 / 0.7))
SUBPAD = 1024
FEAT = 512

_mesh = plsc.VectorSubcoreMesh(core_axis_name="c", subcore_axis_name="s")


def _k1_score(x3_ref, w_ref, xw_ref, t_ref):
    x3 = x3_ref[...]
    w = w_ref[...]
    xw = jnp.sum(x3 * w[None, :, :], axis=2)
    flat = (lax.broadcasted_iota(jnp.int32, (NBLK, 128), 0) * 128
            + lax.broadcasted_iota(jnp.int32, (NBLK, 128), 1))
    xw_ref[...] = jnp.where(flat < N, xw, -1e30)
    nrm = jnp.sqrt(jnp.sum(w * w)) + 1e-12
    t_ref[...] = jnp.full((1, 128), LOGIT_MIN_SCORE, jnp.float32) * nrm


def _k2_mask(xw_hbm, t_hbm, src_hbm, dst_hbm,
             srcc_out, dstc_out, cnt_out, touch_out,
             xw_v, t_v, src_v, dst_v, srcc_v, dstc_v, touch_v, cnt_v):
    c = lax.axis_index("c")
    s = lax.axis_index("s")
    w = c * 16 + s
    base = w * EPW
    pltpu.sync_copy(xw_hbm, xw_v)
    pltpu.sync_copy(t_hbm, t_v)
    pltpu.sync_copy(src_hbm.at[pl.ds(base, EPW)], src_v)
    pltpu.sync_copy(dst_hbm.at[pl.ds(base, EPW)], dst_v)
    t16 = t_v[...]
    ones = jnp.full((16,), 1.0, jnp.float32)
    padv = N + 64 + lax.broadcasted_iota(jnp.int32, (16,), 0)

    def zbody(i, carry):
        touch_v[pl.ds(i * 16, 16)] = jnp.zeros((16,), jnp.float32)
        return carry

    lax.fori_loop(0, NPAD // 16, zbody, 0)

    def body(i, cnt):
        sl = pl.ds(i * 16, 16)
        s16 = src_v[sl]
        d16 = dst_v[sl]
        z = plsc.load_gather(xw_v, [s16]) + plsc.load_gather(xw_v, [d16])
        keep = z > t16
        plsc.store_scatter(touch_v, [s16], ones, mask=keep)
        plsc.store_scatter(touch_v, [d16], ones, mask=keep)
        plsc.store_compressed(srcc_v.at[pl.ds(cnt, 16)], s16, mask=keep)
        plsc.store_compressed(dstc_v.at[pl.ds(cnt, 16)], d16, mask=keep)
        return cnt + plsc.all_reduce_population_count(keep)[0]

    cnt = lax.fori_loop(0, EPW // 16, body, jnp.int32(0))
    for q in range(8):
        srcc_v[pl.ds(cnt + q * 16, 16)] = jnp.zeros((16,), jnp.int32)
        dstc_v[pl.ds(cnt + q * 16, 16)] = padv
    nch = (cnt + 127) // 128
    cnt_v[...] = jnp.broadcast_to(nch, (16,))
    pltpu.sync_copy(srcc_v, srcc_out.at[w])
    pltpu.sync_copy(dstc_v, dstc_out.at[w])
    pltpu.sync_copy(cnt_v, cnt_out.at[w])
    pltpu.sync_copy(touch_v, touch_out.at[w])


def _k3_msg(h_hbm, src3_hbm, dst3_hbm, cnt_hbm, zeros_hbm, out_hbm,
            sidx_v, didx_v, stage_v, cnt_v, acc_sh, sema, semb):
    c = lax.axis_index("c")
    s = lax.axis_index("s")
    w = c * 16 + s
    rows = pl.ds(s * ROWS_PER_TILE, ROWS_PER_TILE)
    pltpu.sync_copy(zeros_hbm, acc_sh.at[rows])
    pltpu.sync_copy(src3_hbm.at[w], sidx_v)
    pltpu.sync_copy(dst3_hbm.at[w], didx_v)
    pltpu.sync_copy(cnt_hbm.at[w], cnt_v)
    nch = cnt_v[...][0]
    plsc.subcore_barrier()

    def body(j, carry):
        da = pltpu.async_copy(h_hbm.at[sidx_v.at[j, pl.ds(0, 64)]],
                              stage_v.at[pl.ds(0, 64)], sema)
        db = pltpu.async_copy(h_hbm.at[sidx_v.at[j, pl.ds(64, 64)]],
                              stage_v.at[pl.ds(64, 64)], semb)
        da.wait()
        db.wait()
        pltpu.sync_copy(stage_v, acc_sh.at[didx_v.at[j]], add=True)
        return carry

    lax.fori_loop(0, nch, body, 0)
    plsc.subcore_barrier()
    pltpu.sync_copy(acc_sh.at[rows], out_hbm.at[c].at[rows])


def _k4_mlp(h_ref, m0_ref, m1_ref, eps_ref, w1_ref, b1_ref, w2_ref, b2_ref,
            o_ref):
    agg = h_ref[...] * eps_ref[...] + m0_ref[...] + m1_ref[...]
    hh = jnp.maximum(
        jnp.dot(agg, w1_ref[...], preferred_element_type=jnp.float32)
        + b1_ref[...], 0.0)
    o_ref[...] = (jnp.dot(hh, w2_ref[...], preferred_element_type=jnp.float32)
                  + b2_ref[...])


def _k5_pool(b3_ref, t_ref, h1_ref, h2_ref, h3_ref, s2g_ref, w1_ref, b1_ref,
             w2_ref, b2_ref, o_ref, x_scr, sums_scr):
    i = pl.program_id(0)
    batch = b3_ref[...].reshape(1, 128)
    m = (jnp.sum(t_ref[...], axis=0, keepdims=True) > 0.0).astype(
        jnp.bfloat16)
    seg = lax.broadcasted_iota(jnp.int32, (SUBPAD, 128), 0)
    sm = ((seg == batch).astype(jnp.float32).astype(jnp.bfloat16)
          * m)
    x_scr[:, 0:128] = h1_ref[...].astype(jnp.bfloat16)
    x_scr[:, 128:256] = h2_ref[...].astype(jnp.bfloat16)
    x_scr[:, 256:384] = h3_ref[...].astype(jnp.bfloat16)
    lane = lax.broadcasted_iota(jnp.int32, (128, 128), 1)
    x_scr[:, 384:512] = (lane == 0).astype(jnp.float32).astype(jnp.bfloat16)

    @pl.when(i == 0)
    def _():
        sums_scr[...] = jnp.zeros((SUBPAD, FEAT), jnp.float32)

    sums_scr[...] += jnp.dot(sm, x_scr[...],
                             preferred_element_type=jnp.float32)

    @pl.when(i == NBLK - 1)
    def _():
        _k6_final(sums_scr, s2g_ref, w1_ref, b1_ref, w2_ref, b2_ref, o_ref)


def _k6_final(sums_ref, s2g_ref, w1_ref, b1_ref, w2_ref, b2_ref, o_ref):
    sums = sums_ref[...]
    s2g = s2g_ref[...].reshape(1, SUBPAD)
    cnt0 = sums[:, 384:512]
    ones_j = jnp.ones((128, 128), jnp.float32)
    den = jnp.maximum(
        jnp.dot(cnt0, ones_j, preferred_element_type=jnp.float32), 1.0)
    gmat = (lax.broadcasted_iota(jnp.int32, (NUM_GRAPHS, SUBPAD), 0)
            == s2g).astype(jnp.float32)
    g1 = jnp.dot(gmat, sums[:, 0:128] / den,
                 preferred_element_type=jnp.float32)
    g2 = jnp.dot(gmat, sums[:, 128:256] / den,
                 preferred_element_type=jnp.float32)
    g3 = jnp.dot(gmat, sums[:, 256:384] / den,
                 preferred_element_type=jnp.float32)
    gcat = jnp.concatenate([g1, g2, g3], axis=1)
    hh = jnp.maximum(
        jnp.dot(gcat, w1_ref[...], preferred_element_type=jnp.float32)
        + b1_ref[...], 0.0)
    o_ref[...] = (jnp.dot(hh, w2_ref[...], preferred_element_type=jnp.float32)
                  + b2_ref[...])


_sc_params = pltpu.CompilerParams(needs_layout_passes=False)

_k2_call = functools.partial(
    pl.kernel, _k2_mask, mesh=_mesh, compiler_params=_sc_params,
    out_type=[jax.ShapeDtypeStruct((NW, EPB), jnp.int32),
              jax.ShapeDtypeStruct((NW, EPB), jnp.int32),
              jax.ShapeDtypeStruct((NW, 16), jnp.int32),
              jax.ShapeDtypeStruct((NW, NPAD), jnp.float32)],
    scratch_types=[pltpu.VMEM((NPAD,), jnp.float32),
                   pltpu.VMEM((16,), jnp.float32),
                   pltpu.VMEM((EPW,), jnp.int32),
                   pltpu.VMEM((EPW,), jnp.int32),
                   pltpu.VMEM((EPB,), jnp.int32),
                   pltpu.VMEM((EPB,), jnp.int32),
                   pltpu.VMEM((NPAD,), jnp.float32),
                   pltpu.VMEM((16,), jnp.int32)])

_k3_call = functools.partial(
    pl.kernel, _k3_msg, mesh=_mesh, compiler_params=_sc_params,
    out_type=jax.ShapeDtypeStruct((2, NPAD, D), jnp.float32),
    scratch_types=[pltpu.VMEM((CHB, 128), jnp.int32),
                   pltpu.VMEM((CHB, 128), jnp.int32),
                   pltpu.VMEM((128, D), jnp.float32),
                   pltpu.VMEM((16,), jnp.int32),
                   pltpu.VMEM_SHARED((NPAD, D), jnp.float32),
                   pltpu.SemaphoreType.DMA,
                   pltpu.SemaphoreType.DMA])


def kernel(x, edge_index, edge_attr, node_to_subgraph, edge_to_subgraph,
           subgraph_to_graph, params):
    f32 = jnp.float32
    src, dst = edge_index[0], edge_index[1]

    xp = jnp.zeros((NPAD, D), f32).at[:N].set(x)
    npadE = EPAD - E
    srcp = jnp.concatenate([src, jnp.full((npadE,), N, jnp.int32)])
    dstp = jnp.concatenate([dst, (jnp.arange(npadE, dtype=jnp.int32) % 64)])
    batchp = jnp.concatenate(
        [node_to_subgraph, jnp.full((NPAD - N,), SUBPAD - 1, jnp.int32)])
    s2gp = jnp.concatenate(
        [subgraph_to_graph,
         jnp.full((SUBPAD - NUM_SUB,), NUM_GRAPHS, jnp.int32)])

    xw79, tvec = pl.pallas_call(
        _k1_score,
        out_shape=[jax.ShapeDtypeStruct((NBLK, 128), f32),
                   jax.ShapeDtypeStruct((1, 128), f32)],
    )(xp.reshape(NBLK, 128, D), params['pool_w'].reshape(1, D))
    xw = xw79.reshape(NPAD)
    t16 = tvec.reshape(128)[:16]

    srcc, dstc, cnts, touch = _k2_call()(xw, t16, srcp, dstp)
    src3 = srcc.reshape(NW, CHB, 128)
    dst3 = dstc.reshape(NW, CHB, 128)

    zeros_tile = jnp.zeros((ROWS_PER_TILE, D), f32)

    nblk = 8
    rblk = NPAD // nblk
    h = xp
    hs = []
    for l in range(N_LAYERS):
        msg2 = _k3_call()(h, src3, dst3, cnts, zeros_tile)
        epsv = jnp.full((1, HID), 1.0 + params['eps_%d' % l], f32)
        h = pl.pallas_call(
            _k4_mlp,
            grid=(nblk,),
            in_specs=[
                pl.BlockSpec((rblk, D), lambda i: (i, 0)),
                pl.BlockSpec((rblk, D), lambda i: (i, 0)),
                pl.BlockSpec((rblk, D), lambda i: (i, 0)),
                pl.BlockSpec((1, HID), lambda i: (0, 0)),
                pl.BlockSpec((D, HID), lambda i: (0, 0)),
                pl.BlockSpec((1, HID), lambda i: (0, 0)),
                pl.BlockSpec((HID, HID), lambda i: (0, 0)),
                pl.BlockSpec((1, HID), lambda i: (0, 0)),
            ],
            out_specs=pl.BlockSpec((rblk, D), lambda i: (i, 0)),
            out_shape=jax.ShapeDtypeStruct((NPAD, HID), f32),
        )(h, msg2[0], msg2[1], epsv, params['W1_%d' % l],
          params['b1_%d' % l].reshape(1, HID), params['W2_%d' % l],
          params['b2_%d' % l].reshape(1, HID))
        hs.append(h)

    w2p = jnp.zeros((HID, 128), f32).at[:, :OUT_DIM].set(params['lin2_W'])
    b2p = jnp.zeros((128,), f32).at[:OUT_DIM].set(params['lin2_b'])
    outp = pl.pallas_call(
        _k5_pool,
        grid=(NBLK,),
        in_specs=[
            pl.BlockSpec((1, 1, 128), lambda i: (i, 0, 0)),
            pl.BlockSpec((NW, 128), lambda i: (0, i)),
            pl.BlockSpec((128, HID), lambda i: (i, 0)),
            pl.BlockSpec((128, HID), lambda i: (i, 0)),
            pl.BlockSpec((128, HID), lambda i: (i, 0)),
            pl.BlockSpec((8, 128), lambda i: (0, 0)),
            pl.BlockSpec((HID * N_LAYERS, HID), lambda i: (0, 0)),
            pl.BlockSpec((1, HID), lambda i: (0, 0)),
            pl.BlockSpec((HID, 128), lambda i: (0, 0)),
            pl.BlockSpec((1, 128), lambda i: (0, 0)),
        ],
        out_specs=pl.BlockSpec((NUM_GRAPHS, 128), lambda i: (0, 0)),
        out_shape=jax.ShapeDtypeStruct((NUM_GRAPHS, 128), f32),
        scratch_shapes=[pltpu.VMEM((128, FEAT), jnp.bfloat16),
                        pltpu.VMEM((SUBPAD, FEAT), f32)],
    )(batchp.reshape(NBLK, 1, 128), touch, hs[0], hs[1], hs[2],
      s2gp.reshape(8, 128), params['lin1_W'],
      params['lin1_b'].reshape(1, HID), w2p, b2p.reshape(1, 128))
    return outp[:, :OUT_DIM]


# speedup vs baseline: 2.0294x; 1.0058x over previous
import functools

import jax
import jax.numpy as jnp
import numpy as np
from jax import lax
from jax.experimental import pallas as pl
from jax.experimental.pallas import tpu as pltpu
from jax.experimental.pallas import tpu_sc as plsc

N = 10000
E = 320000
D = 128
HID = 128
OUT_DIM = 64
N_LAYERS = 3
NUM_SUB = 1000
NUM_GRAPHS = 16

NPAD = 10112
NBLK = NPAD // 128
NW = 32
EPW = 10240
EPAD = NW * EPW
EPB = 10496
CHB = EPB // 128
ROWS_PER_TILE = NPAD // 16
LOGIT_MIN_SCORE = float(np.log(0.3 / 0.7))
SUBPAD = 1024
FEAT = 512

_mesh = plsc.VectorSubcoreMesh(core_axis_name="c", subcore_axis_name="s")


def _k1_score(x3_ref, w_ref, xw_ref, t_ref):
    x3 = x3_ref[...]
    w = w_ref[...]
    xw = jnp.sum(x3 * w[None, :, :], axis=2)
    flat = (lax.broadcasted_iota(jnp.int32, (NBLK, 128), 0) * 128
            + lax.broadcasted_iota(jnp.int32, (NBLK, 128), 1))
    xw_ref[...] = jnp.where(flat < N, xw, -1e30)
    nrm = jnp.sqrt(jnp.sum(w * w)) + 1e-12
    t_ref[...] = jnp.full((1, 128), LOGIT_MIN_SCORE, jnp.float32) * nrm


def _k2_mask(xw_hbm, t_hbm, src_hbm, dst_hbm,
             srcc_out, dstc_out, cnt_out, touch_out,
             xw_v, t_v, src_v, dst_v, srcc_v, dstc_v, touch_v, cnt_v):
    c = lax.axis_index("c")
    s = lax.axis_index("s")
    w = c * 16 + s
    base = w * EPW
    pltpu.sync_copy(xw_hbm, xw_v)
    pltpu.sync_copy(t_hbm, t_v)
    pltpu.sync_copy(src_hbm.at[pl.ds(base, EPW)], src_v)
    pltpu.sync_copy(dst_hbm.at[pl.ds(base, EPW)], dst_v)
    t16 = t_v[...]
    ones = jnp.full((16,), 1.0, jnp.float32)
    padv = N + 64 + lax.broadcasted_iota(jnp.int32, (16,), 0)

    def zbody(i, carry):
        touch_v[pl.ds(i * 16, 16)] = jnp.zeros((16,), jnp.float32)
        return carry

    lax.fori_loop(0, NPAD // 16, zbody, 0)

    def body(i, cnt):
        sl = pl.ds(i * 16, 16)
        s16 = src_v[sl]
        d16 = dst_v[sl]
        z = plsc.load_gather(xw_v, [s16]) + plsc.load_gather(xw_v, [d16])
        keep = z > t16
        plsc.store_scatter(touch_v, [s16], ones, mask=keep)
        plsc.store_scatter(touch_v, [d16], ones, mask=keep)
        plsc.store_compressed(srcc_v.at[pl.ds(cnt, 16)], s16, mask=keep)
        plsc.store_compressed(dstc_v.at[pl.ds(cnt, 16)], d16, mask=keep)
        return cnt + plsc.all_reduce_population_count(keep)[0]

    cnt = lax.fori_loop(0, EPW // 16, body, jnp.int32(0))
    for q in range(8):
        srcc_v[pl.ds(cnt + q * 16, 16)] = jnp.zeros((16,), jnp.int32)
        dstc_v[pl.ds(cnt + q * 16, 16)] = padv
    nch = (cnt + 127) // 128
    cnt_v[...] = jnp.broadcast_to(nch, (16,))
    pltpu.sync_copy(srcc_v, srcc_out.at[w])
    pltpu.sync_copy(dstc_v, dstc_out.at[w])
    pltpu.sync_copy(cnt_v, cnt_out.at[w])
    pltpu.sync_copy(touch_v, touch_out.at[w])


def _k3_msg(h_hbm, src3_hbm, dst3_hbm, cnt_hbm, zeros_hbm, out_hbm,
            sidx_v, didx_v, stage_v, cnt_v, acc_sh, sema, semb):
    c = lax.axis_index("c")
    s = lax.axis_index("s")
    w = c * 16 + s
    rows = pl.ds(s * ROWS_PER_TILE, ROWS_PER_TILE)
    pltpu.sync_copy(zeros_hbm, acc_sh.at[rows])
    pltpu.sync_copy(src3_hbm.at[w], sidx_v)
    pltpu.sync_copy(dst3_hbm.at[w], didx_v)
    pltpu.sync_copy(cnt_hbm.at[w], cnt_v)
    nch = cnt_v[...][0]
    plsc.subcore_barrier()

    def body(j, carry):
        da = pltpu.async_copy(h_hbm.at[sidx_v.at[j, pl.ds(0, 64)]],
                              stage_v.at[pl.ds(0, 64)], sema)
        db = pltpu.async_copy(h_hbm.at[sidx_v.at[j, pl.ds(64, 64)]],
                              stage_v.at[pl.ds(64, 64)], semb)
        da.wait()
        db.wait()
        pltpu.sync_copy(stage_v, acc_sh.at[didx_v.at[j]], add=True)
        return carry

    lax.fori_loop(0, nch, body, 0)
    plsc.subcore_barrier()
    pltpu.sync_copy(acc_sh.at[rows], out_hbm.at[c].at[rows])


def _k4_mlp(h_ref, m0_ref, m1_ref, eps_ref, w1_ref, b1_ref, w2_ref, b2_ref,
            o_ref):
    agg = h_ref[...] * eps_ref[...] + m0_ref[...] + m1_ref[...]
    hh = jnp.maximum(
        jnp.dot(agg, w1_ref[...], preferred_element_type=jnp.float32)
        + b1_ref[...], 0.0)
    o_ref[...] = (jnp.dot(hh, w2_ref[...], preferred_element_type=jnp.float32)
                  + b2_ref[...])


def _k5_pool(b3_ref, t_ref, h1_ref, h2_ref, h3_ref, s2g_ref, w1_ref, b1_ref,
             w2_ref, b2_ref, o_ref, x_scr, sums_scr):
    i = pl.program_id(0)
    batch = b3_ref[...].reshape(1, 128)
    m = (jnp.sum(t_ref[...], axis=0, keepdims=True) > 0.0).astype(
        jnp.bfloat16)
    seg = lax.broadcasted_iota(jnp.int32, (SUBPAD, 128), 0)
    sm = ((seg == batch).astype(jnp.float32).astype(jnp.bfloat16)
          * m)
    x_scr[:, 0:128] = h1_ref[...].astype(jnp.bfloat16)
    x_scr[:, 128:256] = h2_ref[...].astype(jnp.bfloat16)
    x_scr[:, 256:384] = h3_ref[...].astype(jnp.bfloat16)
    lane = lax.broadcasted_iota(jnp.int32, (128, 128), 1)
    x_scr[:, 384:512] = (lane == 0).astype(jnp.float32).astype(jnp.bfloat16)

    @pl.when(i == 0)
    def _():
        sums_scr[...] = jnp.zeros((SUBPAD, FEAT), jnp.float32)

    sums_scr[...] += jnp.dot(sm, x_scr[...],
                             preferred_element_type=jnp.float32)

    @pl.when(i == NBLK - 1)
    def _():
        _k6_final(sums_scr, s2g_ref, w1_ref, b1_ref, w2_ref, b2_ref, o_ref)


def _k6_final(sums_ref, s2g_ref, w1_ref, b1_ref, w2_ref, b2_ref, o_ref):
    sums = sums_ref[...]
    s2g = s2g_ref[...].reshape(1, SUBPAD)
    cnt0 = sums[:, 384:512]
    ones_j = jnp.ones((128, 128), jnp.float32)
    den = jnp.maximum(
        jnp.dot(cnt0, ones_j, preferred_element_type=jnp.float32), 1.0)
    gmat = (lax.broadcasted_iota(jnp.int32, (NUM_GRAPHS, SUBPAD), 0)
            == s2g).astype(jnp.float32)
    g1 = jnp.dot(gmat, sums[:, 0:128] / den,
                 preferred_element_type=jnp.float32)
    g2 = jnp.dot(gmat, sums[:, 128:256] / den,
                 preferred_element_type=jnp.float32)
    g3 = jnp.dot(gmat, sums[:, 256:384] / den,
                 preferred_element_type=jnp.float32)
    gcat = jnp.concatenate([g1, g2, g3], axis=1)
    hh = jnp.maximum(
        jnp.dot(gcat, w1_ref[...], preferred_element_type=jnp.float32)
        + b1_ref[...], 0.0)
    o_ref[...] = (jnp.dot(hh, w2_ref[...], preferred_element_type=jnp.float32)
                  + b2_ref[...])


_sc_params = pltpu.CompilerParams(needs_layout_passes=False)

_k2_call = functools.partial(
    pl.kernel, _k2_mask, mesh=_mesh, compiler_params=_sc_params,
    out_type=[jax.ShapeDtypeStruct((NW, EPB), jnp.int32),
              jax.ShapeDtypeStruct((NW, EPB), jnp.int32),
              jax.ShapeDtypeStruct((NW, 16), jnp.int32),
              jax.ShapeDtypeStruct((NW, NPAD), jnp.float32)],
    scratch_types=[pltpu.VMEM((NPAD,), jnp.float32),
                   pltpu.VMEM((16,), jnp.float32),
                   pltpu.VMEM((EPW,), jnp.int32),
                   pltpu.VMEM((EPW,), jnp.int32),
                   pltpu.VMEM((EPB,), jnp.int32),
                   pltpu.VMEM((EPB,), jnp.int32),
                   pltpu.VMEM((NPAD,), jnp.float32),
                   pltpu.VMEM((16,), jnp.int32)])

_k3_call = functools.partial(
    pl.kernel, _k3_msg, mesh=_mesh, compiler_params=_sc_params,
    out_type=jax.ShapeDtypeStruct((2, NPAD, D), jnp.float32),
    scratch_types=[pltpu.VMEM((CHB, 128), jnp.int32),
                   pltpu.VMEM((CHB, 128), jnp.int32),
                   pltpu.VMEM((128, D), jnp.float32),
                   pltpu.VMEM((16,), jnp.int32),
                   pltpu.VMEM_SHARED((NPAD, D), jnp.float32),
                   pltpu.SemaphoreType.DMA,
                   pltpu.SemaphoreType.DMA])


def kernel(x, edge_index, edge_attr, node_to_subgraph, edge_to_subgraph,
           subgraph_to_graph, params):
    f32 = jnp.float32
    src, dst = edge_index[0], edge_index[1]

    xp = jnp.zeros((NPAD, D), f32).at[:N].set(x)
    npadE = EPAD - E
    srcp = jnp.concatenate([src, jnp.full((npadE,), N, jnp.int32)])
    dstp = jnp.concatenate([dst, (jnp.arange(npadE, dtype=jnp.int32) % 64)])
    batchp = jnp.concatenate(
        [node_to_subgraph, jnp.full((NPAD - N,), SUBPAD - 1, jnp.int32)])
    s2gp = jnp.concatenate(
        [subgraph_to_graph,
         jnp.full((SUBPAD - NUM_SUB,), NUM_GRAPHS, jnp.int32)])

    xw79, tvec = pl.pallas_call(
        _k1_score,
        out_shape=[jax.ShapeDtypeStruct((NBLK, 128), f32),
                   jax.ShapeDtypeStruct((1, 128), f32)],
    )(xp.reshape(NBLK, 128, D), params['pool_w'].reshape(1, D))
    xw = xw79.reshape(NPAD)
    t16 = tvec.reshape(128)[:16]

    srcc, dstc, cnts, touch = _k2_call()(xw, t16, srcp, dstp)
    src3 = srcc.reshape(NW, CHB, 128)
    dst3 = dstc.reshape(NW, CHB, 128)

    zeros_tile = jnp.zeros((ROWS_PER_TILE, D), f32)

    nblk = 4
    rblk = NPAD // nblk
    h = xp
    hs = []
    for l in range(N_LAYERS):
        msg2 = _k3_call()(h, src3, dst3, cnts, zeros_tile)
        epsv = jnp.full((1, HID), 1.0 + params['eps_%d' % l], f32)
        h = pl.pallas_call(
            _k4_mlp,
            grid=(nblk,),
            in_specs=[
                pl.BlockSpec((rblk, D), lambda i: (i, 0)),
                pl.BlockSpec((rblk, D), lambda i: (i, 0)),
                pl.BlockSpec((rblk, D), lambda i: (i, 0)),
                pl.BlockSpec((1, HID), lambda i: (0, 0)),
                pl.BlockSpec((D, HID), lambda i: (0, 0)),
                pl.BlockSpec((1, HID), lambda i: (0, 0)),
                pl.BlockSpec((HID, HID), lambda i: (0, 0)),
                pl.BlockSpec((1, HID), lambda i: (0, 0)),
            ],
            out_specs=pl.BlockSpec((rblk, D), lambda i: (i, 0)),
            out_shape=jax.ShapeDtypeStruct((NPAD, HID), f32),
        )(h, msg2[0], msg2[1], epsv, params['W1_%d' % l],
          params['b1_%d' % l].reshape(1, HID), params['W2_%d' % l],
          params['b2_%d' % l].reshape(1, HID))
        hs.append(h)

    w2p = jnp.zeros((HID, 128), f32).at[:, :OUT_DIM].set(params['lin2_W'])
    b2p = jnp.zeros((128,), f32).at[:OUT_DIM].set(params['lin2_b'])
    outp = pl.pallas_call(
        _k5_pool,
        grid=(NBLK,),
        in_specs=[
            pl.BlockSpec((1, 1, 128), lambda i: (i, 0, 0)),
            pl.BlockSpec((NW, 128), lambda i: (0, i)),
            pl.BlockSpec((128, HID), lambda i: (i, 0)),
            pl.BlockSpec((128, HID), lambda i: (i, 0)),
            pl.BlockSpec((128, HID), lambda i: (i, 0)),
            pl.BlockSpec((8, 128), lambda i: (0, 0)),
            pl.BlockSpec((HID * N_LAYERS, HID), lambda i: (0, 0)),
            pl.BlockSpec((1, HID), lambda i: (0, 0)),
            pl.BlockSpec((HID, 128), lambda i: (0, 0)),
            pl.BlockSpec((1, 128), lambda i: (0, 0)),
        ],
        out_specs=pl.BlockSpec((NUM_GRAPHS, 128), lambda i: (0, 0)),
        out_shape=jax.ShapeDtypeStruct((NUM_GRAPHS, 128), f32),
        scratch_shapes=[pltpu.VMEM((128, FEAT), jnp.bfloat16),
                        pltpu.VMEM((SUBPAD, FEAT), f32)],
    )(batchp.reshape(NBLK, 1, 128), touch, hs[0], hs[1], hs[2],
      s2gp.reshape(8, 128), params['lin1_W'],
      params['lin1_b'].reshape(1, HID), w2p, b2p.reshape(1, 128))
    return outp[:, :OUT_DIM]


# speedup vs baseline: 2.0479x; 1.0091x over previous
import functools

import jax
import jax.numpy as jnp
import numpy as np
from jax import lax
from jax.experimental import pallas as pl
from jax.experimental.pallas import tpu as pltpu
from jax.experimental.pallas import tpu_sc as plsc

N = 10000
E = 320000
D = 128
HID = 128
OUT_DIM = 64
N_LAYERS = 3
NUM_SUB = 1000
NUM_GRAPHS = 16

NPAD = 10112
NBLK = NPAD // 128
NW = 32
EPW = 10240
EPAD = NW * EPW
EPB = 10496
CHB = EPB // 128
ROWS_PER_TILE = NPAD // 16
LOGIT_MIN_SCORE = float(np.log(0.3 / 0.7))
SUBPAD = 1024
FEAT = 512

_mesh = plsc.VectorSubcoreMesh(core_axis_name="c", subcore_axis_name="s")


def _k1_score(x3_ref, w_ref, xw_ref, t_ref):
    x3 = x3_ref[...]
    w = w_ref[...]
    xw = jnp.sum(x3 * w[None, :, :], axis=2)
    flat = (lax.broadcasted_iota(jnp.int32, (NBLK, 128), 0) * 128
            + lax.broadcasted_iota(jnp.int32, (NBLK, 128), 1))
    xw_ref[...] = jnp.where(flat < N, xw, -1e30)
    nrm = jnp.sqrt(jnp.sum(w * w)) + 1e-12
    t_ref[...] = jnp.full((1, 128), LOGIT_MIN_SCORE, jnp.float32) * nrm


def _k2_mask(xw_hbm, t_hbm, src_hbm, dst_hbm,
             srcc_out, dstc_out, cnt_out, touch_out,
             xw_v, t_v, src_v, dst_v, srcc_v, dstc_v, touch_v, cnt_v):
    c = lax.axis_index("c")
    s = lax.axis_index("s")
    w = c * 16 + s
    base = w * EPW
    pltpu.sync_copy(xw_hbm, xw_v)
    pltpu.sync_copy(t_hbm, t_v)
    pltpu.sync_copy(src_hbm.at[pl.ds(base, EPW)], src_v)
    pltpu.sync_copy(dst_hbm.at[pl.ds(base, EPW)], dst_v)
    t16 = t_v[...]
    ones = jnp.full((16,), 1.0, jnp.float32)
    padv = N + 64 + lax.broadcasted_iota(jnp.int32, (16,), 0)

    def zbody(i, carry):
        touch_v[pl.ds(i * 16, 16)] = jnp.zeros((16,), jnp.float32)
        return carry

    lax.fori_loop(0, NPAD // 16, zbody, 0)

    def body(i, cnt):
        sl = pl.ds(i * 16, 16)
        s16 = src_v[sl]
        d16 = dst_v[sl]
        z = plsc.load_gather(xw_v, [s16]) + plsc.load_gather(xw_v, [d16])
        keep = z > t16
        plsc.store_scatter(touch_v, [s16], ones, mask=keep)
        plsc.store_scatter(touch_v, [d16], ones, mask=keep)
        plsc.store_compressed(srcc_v.at[pl.ds(cnt, 16)], s16, mask=keep)
        plsc.store_compressed(dstc_v.at[pl.ds(cnt, 16)], d16, mask=keep)
        return cnt + plsc.all_reduce_population_count(keep)[0]

    cnt = lax.fori_loop(0, EPW // 16, body, jnp.int32(0))
    for q in range(8):
        srcc_v[pl.ds(cnt + q * 16, 16)] = jnp.zeros((16,), jnp.int32)
        dstc_v[pl.ds(cnt + q * 16, 16)] = padv
    nch = (cnt + 127) // 128
    cnt_v[...] = jnp.broadcast_to(nch, (16,))
    pltpu.sync_copy(srcc_v, srcc_out.at[w])
    pltpu.sync_copy(dstc_v, dstc_out.at[w])
    pltpu.sync_copy(cnt_v, cnt_out.at[w])
    pltpu.sync_copy(touch_v, touch_out.at[w])


def _k3_msg(h_hbm, src3_hbm, dst3_hbm, cnt_hbm, zeros_hbm, out_hbm,
            sidx_v, didx_v, stage_v, cnt_v, acc_sh, sema, semb):
    c = lax.axis_index("c")
    s = lax.axis_index("s")
    w = c * 16 + s
    rows = pl.ds(s * ROWS_PER_TILE, ROWS_PER_TILE)
    dz = pltpu.async_copy(zeros_hbm, acc_sh.at[rows], sema)
    ds_ = pltpu.async_copy(src3_hbm.at[w], sidx_v, semb)
    pltpu.sync_copy(dst3_hbm.at[w], didx_v)
    pltpu.sync_copy(cnt_hbm.at[w], cnt_v)
    nch = cnt_v[...][0]
    ds_.wait()
    dz.wait()
    plsc.subcore_barrier()

    def body(j, carry):
        da = pltpu.async_copy(h_hbm.at[sidx_v.at[j, pl.ds(0, 64)]],
                              stage_v.at[pl.ds(0, 64)], sema)
        db = pltpu.async_copy(h_hbm.at[sidx_v.at[j, pl.ds(64, 64)]],
                              stage_v.at[pl.ds(64, 64)], semb)
        da.wait()
        db.wait()
        pltpu.sync_copy(stage_v, acc_sh.at[didx_v.at[j]], add=True)
        return carry

    lax.fori_loop(0, nch, body, 0)
    plsc.subcore_barrier()
    pltpu.sync_copy(acc_sh.at[rows], out_hbm.at[c].at[rows])


def _k4_mlp(h_ref, m0_ref, m1_ref, eps_ref, w1_ref, b1_ref, w2_ref, b2_ref,
            o_ref):
    agg = h_ref[...] * eps_ref[...] + m0_ref[...] + m1_ref[...]
    hh = jnp.maximum(
        jnp.dot(agg, w1_ref[...], preferred_element_type=jnp.float32)
        + b1_ref[...], 0.0)
    o_ref[...] = (jnp.dot(hh, w2_ref[...], preferred_element_type=jnp.float32)
                  + b2_ref[...])


def _k5_pool(b3_ref, t_ref, h1_ref, h2_ref, h3_ref, s2g_ref, w1_ref, b1_ref,
             w2_ref, b2_ref, o_ref, x_scr, sums_scr):
    i = pl.program_id(0)
    batch = b3_ref[...].reshape(1, 128)
    m = (jnp.sum(t_ref[...], axis=0, keepdims=True) > 0.0).astype(
        jnp.bfloat16)
    seg = lax.broadcasted_iota(jnp.int32, (SUBPAD, 128), 0)
    sm = ((seg == batch).astype(jnp.float32).astype(jnp.bfloat16)
          * m)
    x_scr[:, 0:128] = h1_ref[...].astype(jnp.bfloat16)
    x_scr[:, 128:256] = h2_ref[...].astype(jnp.bfloat16)
    x_scr[:, 256:384] = h3_ref[...].astype(jnp.bfloat16)
    lane = lax.broadcasted_iota(jnp.int32, (128, 128), 1)
    x_scr[:, 384:512] = (lane == 0).astype(jnp.float32).astype(jnp.bfloat16)

    @pl.when(i == 0)
    def _():
        sums_scr[...] = jnp.zeros((SUBPAD, FEAT), jnp.float32)

    sums_scr[...] += jnp.dot(sm, x_scr[...],
                             preferred_element_type=jnp.float32)

    @pl.when(i == NBLK - 1)
    def _():
        _k6_final(sums_scr, s2g_ref, w1_ref, b1_ref, w2_ref, b2_ref, o_ref)


def _k6_final(sums_ref, s2g_ref, w1_ref, b1_ref, w2_ref, b2_ref, o_ref):
    sums = sums_ref[...]
    s2g = s2g_ref[...].reshape(1, SUBPAD)
    cnt0 = sums[:, 384:512]
    ones_j = jnp.ones((128, 128), jnp.float32)
    den = jnp.maximum(
        jnp.dot(cnt0, ones_j, preferred_element_type=jnp.float32), 1.0)
    gmat = (lax.broadcasted_iota(jnp.int32, (NUM_GRAPHS, SUBPAD), 0)
            == s2g).astype(jnp.float32)
    g1 = jnp.dot(gmat, sums[:, 0:128] / den,
                 preferred_element_type=jnp.float32)
    g2 = jnp.dot(gmat, sums[:, 128:256] / den,
                 preferred_element_type=jnp.float32)
    g3 = jnp.dot(gmat, sums[:, 256:384] / den,
                 preferred_element_type=jnp.float32)
    gcat = jnp.concatenate([g1, g2, g3], axis=1)
    hh = jnp.maximum(
        jnp.dot(gcat, w1_ref[...], preferred_element_type=jnp.float32)
        + b1_ref[...], 0.0)
    o_ref[...] = (jnp.dot(hh, w2_ref[...], preferred_element_type=jnp.float32)
                  + b2_ref[...])


_sc_params = pltpu.CompilerParams(needs_layout_passes=False)

_k2_call = functools.partial(
    pl.kernel, _k2_mask, mesh=_mesh, compiler_params=_sc_params,
    out_type=[jax.ShapeDtypeStruct((NW, EPB), jnp.int32),
              jax.ShapeDtypeStruct((NW, EPB), jnp.int32),
              jax.ShapeDtypeStruct((NW, 16), jnp.int32),
              jax.ShapeDtypeStruct((NW, NPAD), jnp.float32)],
    scratch_types=[pltpu.VMEM((NPAD,), jnp.float32),
                   pltpu.VMEM((16,), jnp.float32),
                   pltpu.VMEM((EPW,), jnp.int32),
                   pltpu.VMEM((EPW,), jnp.int32),
                   pltpu.VMEM((EPB,), jnp.int32),
                   pltpu.VMEM((EPB,), jnp.int32),
                   pltpu.VMEM((NPAD,), jnp.float32),
                   pltpu.VMEM((16,), jnp.int32)])

_k3_call = functools.partial(
    pl.kernel, _k3_msg, mesh=_mesh, compiler_params=_sc_params,
    out_type=jax.ShapeDtypeStruct((2, NPAD, D), jnp.float32),
    scratch_types=[pltpu.VMEM((CHB, 128), jnp.int32),
                   pltpu.VMEM((CHB, 128), jnp.int32),
                   pltpu.VMEM((128, D), jnp.float32),
                   pltpu.VMEM((16,), jnp.int32),
                   pltpu.VMEM_SHARED((NPAD, D), jnp.float32),
                   pltpu.SemaphoreType.DMA,
                   pltpu.SemaphoreType.DMA])


def kernel(x, edge_index, edge_attr, node_to_subgraph, edge_to_subgraph,
           subgraph_to_graph, params):
    f32 = jnp.float32
    src, dst = edge_index[0], edge_index[1]

    xp = jnp.zeros((NPAD, D), f32).at[:N].set(x)
    npadE = EPAD - E
    srcp = jnp.concatenate([src, jnp.full((npadE,), N, jnp.int32)])
    dstp = jnp.concatenate([dst, (jnp.arange(npadE, dtype=jnp.int32) % 64)])
    batchp = jnp.concatenate(
        [node_to_subgraph, jnp.full((NPAD - N,), SUBPAD - 1, jnp.int32)])
    s2gp = jnp.concatenate(
        [subgraph_to_graph,
         jnp.full((SUBPAD - NUM_SUB,), NUM_GRAPHS, jnp.int32)])

    xw79, tvec = pl.pallas_call(
        _k1_score,
        out_shape=[jax.ShapeDtypeStruct((NBLK, 128), f32),
                   jax.ShapeDtypeStruct((1, 128), f32)],
    )(xp.reshape(NBLK, 128, D), params['pool_w'].reshape(1, D))
    xw = xw79.reshape(NPAD)
    t16 = tvec.reshape(128)[:16]

    srcc, dstc, cnts, touch = _k2_call()(xw, t16, srcp, dstp)
    src3 = srcc.reshape(NW, CHB, 128)
    dst3 = dstc.reshape(NW, CHB, 128)

    zeros_tile = jnp.zeros((ROWS_PER_TILE, D), f32)

    nblk = 4
    rblk = NPAD // nblk
    h = xp
    hs = []
    for l in range(N_LAYERS):
        msg2 = _k3_call()(h, src3, dst3, cnts, zeros_tile)
        epsv = jnp.full((1, HID), 1.0 + params['eps_%d' % l], f32)
        h = pl.pallas_call(
            _k4_mlp,
            grid=(nblk,),
            in_specs=[
                pl.BlockSpec((rblk, D), lambda i: (i, 0)),
                pl.BlockSpec((rblk, D), lambda i: (i, 0)),
                pl.BlockSpec((rblk, D), lambda i: (i, 0)),
                pl.BlockSpec((1, HID), lambda i: (0, 0)),
                pl.BlockSpec((D, HID), lambda i: (0, 0)),
                pl.BlockSpec((1, HID), lambda i: (0, 0)),
                pl.BlockSpec((HID, HID), lambda i: (0, 0)),
                pl.BlockSpec((1, HID), lambda i: (0, 0)),
            ],
            out_specs=pl.BlockSpec((rblk, D), lambda i: (i, 0)),
            out_shape=jax.ShapeDtypeStruct((NPAD, HID), f32),
        )(h, msg2[0], msg2[1], epsv, params['W1_%d' % l],
          params['b1_%d' % l].reshape(1, HID), params['W2_%d' % l],
          params['b2_%d' % l].reshape(1, HID))
        hs.append(h)

    w2p = jnp.zeros((HID, 128), f32).at[:, :OUT_DIM].set(params['lin2_W'])
    b2p = jnp.zeros((128,), f32).at[:OUT_DIM].set(params['lin2_b'])
    outp = pl.pallas_call(
        _k5_pool,
        grid=(NBLK,),
        in_specs=[
            pl.BlockSpec((1, 1, 128), lambda i: (i, 0, 0)),
            pl.BlockSpec((NW, 128), lambda i: (0, i)),
            pl.BlockSpec((128, HID), lambda i: (i, 0)),
            pl.BlockSpec((128, HID), lambda i: (i, 0)),
            pl.BlockSpec((128, HID), lambda i: (i, 0)),
            pl.BlockSpec((8, 128), lambda i: (0, 0)),
            pl.BlockSpec((HID * N_LAYERS, HID), lambda i: (0, 0)),
            pl.BlockSpec((1, HID), lambda i: (0, 0)),
            pl.BlockSpec((HID, 128), lambda i: (0, 0)),
            pl.BlockSpec((1, 128), lambda i: (0, 0)),
        ],
        out_specs=pl.BlockSpec((NUM_GRAPHS, 128), lambda i: (0, 0)),
        out_shape=jax.ShapeDtypeStruct((NUM_GRAPHS, 128), f32),
        scratch_shapes=[pltpu.VMEM((128, FEAT), jnp.bfloat16),
                        pltpu.VMEM((SUBPAD, FEAT), f32)],
    )(batchp.reshape(NBLK, 1, 128), touch, hs[0], hs[1], hs[2],
      s2gp.reshape(8, 128), params['lin1_W'],
      params['lin1_b'].reshape(1, HID), w2p, b2p.reshape(1, 128))
    return outp[:, :OUT_DIM]


# speedup vs baseline: 2.0567x; 1.0043x over previous
import functools

import jax
import jax.numpy as jnp
import numpy as np
from jax import lax
from jax.experimental import pallas as pl
from jax.experimental.pallas import tpu as pltpu
from jax.experimental.pallas import tpu_sc as plsc

N = 10000
E = 320000
D = 128
HID = 128
OUT_DIM = 64
N_LAYERS = 3
NUM_SUB = 1000
NUM_GRAPHS = 16

NPAD = 10112
NBLK = NPAD // 128
NW = 32
EPW = 10240
EPAD = NW * EPW
EPB = 10496
CHB = EPB // 128
ROWS_PER_TILE = NPAD // 16
LOGIT_MIN_SCORE = float(np.log(0.3 / 0.7))
SUBPAD = 1024
FEAT = 512

_mesh = plsc.VectorSubcoreMesh(core_axis_name="c", subcore_axis_name="s")


def _k1_score(x3_ref, w_ref, xw_ref, t_ref):
    x3 = x3_ref[...]
    w = w_ref[...]
    xw = jnp.sum(x3 * w[None, :, :], axis=2)
    flat = (lax.broadcasted_iota(jnp.int32, (NBLK, 128), 0) * 128
            + lax.broadcasted_iota(jnp.int32, (NBLK, 128), 1))
    xw_ref[...] = jnp.where(flat < N, xw, -1e30)
    nrm = jnp.sqrt(jnp.sum(w * w)) + 1e-12
    t_ref[...] = jnp.full((1, 128), LOGIT_MIN_SCORE, jnp.float32) * nrm


def _k2_mask(xw_hbm, t_hbm, src_hbm, dst_hbm,
             srcc_out, dstc_out, cnt_out, touch_out,
             xw_v, t_v, src_v, dst_v, srcc_v, dstc_v, touch_v, cnt_v,
             ksem, ssem, dsem):
    c = lax.axis_index("c")
    s = lax.axis_index("s")
    w = c * 16 + s
    base = w * EPW
    dx = pltpu.async_copy(xw_hbm, xw_v, ksem)
    ds_ = pltpu.async_copy(src_hbm.at[pl.ds(base, EPW)], src_v, ssem)
    dd = pltpu.async_copy(dst_hbm.at[pl.ds(base, EPW)], dst_v, dsem)
    pltpu.sync_copy(t_hbm, t_v)
    t16 = t_v[...]
    ones = jnp.full((16,), 1.0, jnp.float32)
    padv = N + 64 + lax.broadcasted_iota(jnp.int32, (16,), 0)

    def zbody(i, carry):
        touch_v[pl.ds(i * 16, 16)] = jnp.zeros((16,), jnp.float32)
        return carry

    lax.fori_loop(0, NPAD // 16, zbody, 0)
    dx.wait()
    ds_.wait()
    dd.wait()

    def body(i, cnt):
        sl = pl.ds(i * 16, 16)
        s16 = src_v[sl]
        d16 = dst_v[sl]
        z = plsc.load_gather(xw_v, [s16]) + plsc.load_gather(xw_v, [d16])
        keep = z > t16
        plsc.store_scatter(touch_v, [s16], ones, mask=keep)
        plsc.store_scatter(touch_v, [d16], ones, mask=keep)
        plsc.store_compressed(srcc_v.at[pl.ds(cnt, 16)], s16, mask=keep)
        plsc.store_compressed(dstc_v.at[pl.ds(cnt, 16)], d16, mask=keep)
        return cnt + plsc.all_reduce_population_count(keep)[0]

    cnt = lax.fori_loop(0, EPW // 16, body, jnp.int32(0))
    for q in range(8):
        srcc_v[pl.ds(cnt + q * 16, 16)] = jnp.zeros((16,), jnp.int32)
        dstc_v[pl.ds(cnt + q * 16, 16)] = padv
    nch = (cnt + 127) // 128
    cnt_v[...] = jnp.broadcast_to(nch, (16,))
    pltpu.sync_copy(srcc_v, srcc_out.at[w])
    pltpu.sync_copy(dstc_v, dstc_out.at[w])
    pltpu.sync_copy(cnt_v, cnt_out.at[w])
    pltpu.sync_copy(touch_v, touch_out.at[w])


def _k3_msg(h_hbm, src3_hbm, dst3_hbm, cnt_hbm, zeros_hbm, out_hbm,
            sidx_v, didx_v, stage_v, cnt_v, acc_sh, sema, semb):
    c = lax.axis_index("c")
    s = lax.axis_index("s")
    w = c * 16 + s
    rows = pl.ds(s * ROWS_PER_TILE, ROWS_PER_TILE)
    dz = pltpu.async_copy(zeros_hbm, acc_sh.at[rows], sema)
    ds_ = pltpu.async_copy(src3_hbm.at[w], sidx_v, semb)
    pltpu.sync_copy(dst3_hbm.at[w], didx_v)
    pltpu.sync_copy(cnt_hbm.at[w], cnt_v)
    nch = cnt_v[...][0]
    ds_.wait()
    dz.wait()
    plsc.subcore_barrier()

    def body(j, carry):
        da = pltpu.async_copy(h_hbm.at[sidx_v.at[j, pl.ds(0, 64)]],
                              stage_v.at[pl.ds(0, 64)], sema)
        db = pltpu.async_copy(h_hbm.at[sidx_v.at[j, pl.ds(64, 64)]],
                              stage_v.at[pl.ds(64, 64)], semb)
        da.wait()
        db.wait()
        pltpu.sync_copy(stage_v, acc_sh.at[didx_v.at[j]], add=True)
        return carry

    lax.fori_loop(0, nch, body, 0)
    plsc.subcore_barrier()
    pltpu.sync_copy(acc_sh.at[rows], out_hbm.at[c].at[rows])


def _k4_mlp(h_ref, m0_ref, m1_ref, eps_ref, w1_ref, b1_ref, w2_ref, b2_ref,
            o_ref):
    agg = h_ref[...] * eps_ref[...] + m0_ref[...] + m1_ref[...]
    hh = jnp.maximum(
        jnp.dot(agg, w1_ref[...], preferred_element_type=jnp.float32)
        + b1_ref[...], 0.0)
    o_ref[...] = (jnp.dot(hh, w2_ref[...], preferred_element_type=jnp.float32)
                  + b2_ref[...])


def _k5_pool(b3_ref, t_ref, h1_ref, h2_ref, h3_ref, s2g_ref, w1_ref, b1_ref,
             w2_ref, b2_ref, o_ref, x_scr, sums_scr):
    i = pl.program_id(0)
    batch = b3_ref[...].reshape(1, 128)
    m = (jnp.sum(t_ref[...], axis=0, keepdims=True) > 0.0).astype(
        jnp.bfloat16)
    seg = lax.broadcasted_iota(jnp.int32, (SUBPAD, 128), 0)
    sm = ((seg == batch).astype(jnp.float32).astype(jnp.bfloat16)
          * m)
    x_scr[:, 0:128] = h1_ref[...].astype(jnp.bfloat16)
    x_scr[:, 128:256] = h2_ref[...].astype(jnp.bfloat16)
    x_scr[:, 256:384] = h3_ref[...].astype(jnp.bfloat16)
    lane = lax.broadcasted_iota(jnp.int32, (128, 128), 1)
    x_scr[:, 384:512] = (lane == 0).astype(jnp.float32).astype(jnp.bfloat16)

    @pl.when(i == 0)
    def _():
        sums_scr[...] = jnp.zeros((SUBPAD, FEAT), jnp.float32)

    sums_scr[...] += jnp.dot(sm, x_scr[...],
                             preferred_element_type=jnp.float32)

    @pl.when(i == NBLK - 1)
    def _():
        _k6_final(sums_scr, s2g_ref, w1_ref, b1_ref, w2_ref, b2_ref, o_ref)


def _k6_final(sums_ref, s2g_ref, w1_ref, b1_ref, w2_ref, b2_ref, o_ref):
    sums = sums_ref[...]
    s2g = s2g_ref[...].reshape(1, SUBPAD)
    cnt0 = sums[:, 384:512]
    ones_j = jnp.ones((128, 128), jnp.float32)
    den = jnp.maximum(
        jnp.dot(cnt0, ones_j, preferred_element_type=jnp.float32), 1.0)
    gmat = (lax.broadcasted_iota(jnp.int32, (NUM_GRAPHS, SUBPAD), 0)
            == s2g).astype(jnp.float32)
    g1 = jnp.dot(gmat, sums[:, 0:128] / den,
                 preferred_element_type=jnp.float32)
    g2 = jnp.dot(gmat, sums[:, 128:256] / den,
                 preferred_element_type=jnp.float32)
    g3 = jnp.dot(gmat, sums[:, 256:384] / den,
                 preferred_element_type=jnp.float32)
    gcat = jnp.concatenate([g1, g2, g3], axis=1)
    hh = jnp.maximum(
        jnp.dot(gcat, w1_ref[...], preferred_element_type=jnp.float32)
        + b1_ref[...], 0.0)
    o_ref[...] = (jnp.dot(hh, w2_ref[...], preferred_element_type=jnp.float32)
                  + b2_ref[...])


_sc_params = pltpu.CompilerParams(needs_layout_passes=False)

_k2_call = functools.partial(
    pl.kernel, _k2_mask, mesh=_mesh, compiler_params=_sc_params,
    out_type=[jax.ShapeDtypeStruct((NW, EPB), jnp.int32),
              jax.ShapeDtypeStruct((NW, EPB), jnp.int32),
              jax.ShapeDtypeStruct((NW, 16), jnp.int32),
              jax.ShapeDtypeStruct((NW, NPAD), jnp.float32)],
    scratch_types=[pltpu.VMEM((NPAD,), jnp.float32),
                   pltpu.VMEM((16,), jnp.float32),
                   pltpu.VMEM((EPW,), jnp.int32),
                   pltpu.VMEM((EPW,), jnp.int32),
                   pltpu.VMEM((EPB,), jnp.int32),
                   pltpu.VMEM((EPB,), jnp.int32),
                   pltpu.VMEM((NPAD,), jnp.float32),
                   pltpu.VMEM((16,), jnp.int32),
                   pltpu.SemaphoreType.DMA,
                   pltpu.SemaphoreType.DMA,
                   pltpu.SemaphoreType.DMA])

_k3_call = functools.partial(
    pl.kernel, _k3_msg, mesh=_mesh, compiler_params=_sc_params,
    out_type=jax.ShapeDtypeStruct((2, NPAD, D), jnp.float32),
    scratch_types=[pltpu.VMEM((CHB, 128), jnp.int32),
                   pltpu.VMEM((CHB, 128), jnp.int32),
                   pltpu.VMEM((128, D), jnp.float32),
                   pltpu.VMEM((16,), jnp.int32),
                   pltpu.VMEM_SHARED((NPAD, D), jnp.float32),
                   pltpu.SemaphoreType.DMA,
                   pltpu.SemaphoreType.DMA])


def kernel(x, edge_index, edge_attr, node_to_subgraph, edge_to_subgraph,
           subgraph_to_graph, params):
    f32 = jnp.float32
    src, dst = edge_index[0], edge_index[1]

    xp = jnp.zeros((NPAD, D), f32).at[:N].set(x)
    npadE = EPAD - E
    srcp = jnp.concatenate([src, jnp.full((npadE,), N, jnp.int32)])
    dstp = jnp.concatenate([dst, (jnp.arange(npadE, dtype=jnp.int32) % 64)])
    batchp = jnp.concatenate(
        [node_to_subgraph, jnp.full((NPAD - N,), SUBPAD - 1, jnp.int32)])
    s2gp = jnp.concatenate(
        [subgraph_to_graph,
         jnp.full((SUBPAD - NUM_SUB,), NUM_GRAPHS, jnp.int32)])

    xw79, tvec = pl.pallas_call(
        _k1_score,
        out_shape=[jax.ShapeDtypeStruct((NBLK, 128), f32),
                   jax.ShapeDtypeStruct((1, 128), f32)],
    )(xp.reshape(NBLK, 128, D), params['pool_w'].reshape(1, D))
    xw = xw79.reshape(NPAD)
    t16 = tvec.reshape(128)[:16]

    srcc, dstc, cnts, touch = _k2_call()(xw, t16, srcp, dstp)
    src3 = srcc.reshape(NW, CHB, 128)
    dst3 = dstc.reshape(NW, CHB, 128)

    zeros_tile = jnp.zeros((ROWS_PER_TILE, D), f32)

    nblk = 4
    rblk = NPAD // nblk
    h = xp
    hs = []
    for l in range(N_LAYERS):
        msg2 = _k3_call()(h, src3, dst3, cnts, zeros_tile)
        epsv = jnp.full((1, HID), 1.0 + params['eps_%d' % l], f32)
        h = pl.pallas_call(
            _k4_mlp,
            grid=(nblk,),
            in_specs=[
                pl.BlockSpec((rblk, D), lambda i: (i, 0)),
                pl.BlockSpec((rblk, D), lambda i: (i, 0)),
                pl.BlockSpec((rblk, D), lambda i: (i, 0)),
                pl.BlockSpec((1, HID), lambda i: (0, 0)),
                pl.BlockSpec((D, HID), lambda i: (0, 0)),
                pl.BlockSpec((1, HID), lambda i: (0, 0)),
                pl.BlockSpec((HID, HID), lambda i: (0, 0)),
                pl.BlockSpec((1, HID), lambda i: (0, 0)),
            ],
            out_specs=pl.BlockSpec((rblk, D), lambda i: (i, 0)),
            out_shape=jax.ShapeDtypeStruct((NPAD, HID), f32),
        )(h, msg2[0], msg2[1], epsv, params['W1_%d' % l],
          params['b1_%d' % l].reshape(1, HID), params['W2_%d' % l],
          params['b2_%d' % l].reshape(1, HID))
        hs.append(h)

    w2p = jnp.zeros((HID, 128), f32).at[:, :OUT_DIM].set(params['lin2_W'])
    b2p = jnp.zeros((128,), f32).at[:OUT_DIM].set(params['lin2_b'])
    outp = pl.pallas_call(
        _k5_pool,
        grid=(NBLK,),
        in_specs=[
            pl.BlockSpec((1, 1, 128), lambda i: (i, 0, 0)),
            pl.BlockSpec((NW, 128), lambda i: (0, i)),
            pl.BlockSpec((128, HID), lambda i: (i, 0)),
            pl.BlockSpec((128, HID), lambda i: (i, 0)),
            pl.BlockSpec((128, HID), lambda i: (i, 0)),
            pl.BlockSpec((8, 128), lambda i: (0, 0)),
            pl.BlockSpec((HID * N_LAYERS, HID), lambda i: (0, 0)),
            pl.BlockSpec((1, HID), lambda i: (0, 0)),
            pl.BlockSpec((HID, 128), lambda i: (0, 0)),
            pl.BlockSpec((1, 128), lambda i: (0, 0)),
        ],
        out_specs=pl.BlockSpec((NUM_GRAPHS, 128), lambda i: (0, 0)),
        out_shape=jax.ShapeDtypeStruct((NUM_GRAPHS, 128), f32),
        scratch_shapes=[pltpu.VMEM((128, FEAT), jnp.bfloat16),
                        pltpu.VMEM((SUBPAD, FEAT), f32)],
    )(batchp.reshape(NBLK, 1, 128), touch, hs[0], hs[1], hs[2],
      s2gp.reshape(8, 128), params['lin1_W'],
      params['lin1_b'].reshape(1, HID), w2p, b2p.reshape(1, 128))
    return outp[:, :OUT_DIM]
